# Initial kernel scaffold; baseline (speedup 1.0000x reference)
import jax
import jax.numpy as jnp
from jax.experimental import pallas as pl


def kernel(pos, h, edge_attr, clash_feat, edge_index, node_time_emb, edge_time_emb, params):
    raise NotImplementedError("write your pallas kernel here")


# speedup vs baseline: 10.6965x; 10.6965x over previous
import functools

import jax
import jax.numpy as jnp
import numpy as np
from jax import lax
from jax.experimental import pallas as pl
from jax.experimental.pallas import tpu as pltpu
from jax.experimental.pallas import tpu_sc as plsc

F32 = jnp.float32

NODE_DIM = 128
EDGE_DIM = 16
DIST_DIM = 64
HEADS = 8
HEAD_C = 16
WROW = 144

_OFFS = np.linspace(0.0, 15.0, DIST_DIM).astype(np.float32)
_COEFF = float(-0.5 / (_OFFS[1] - _OFFS[0]) ** 2)
_HM = np.kron(np.eye(HEADS, dtype=np.float32), np.ones((HEAD_C, 1), np.float32))
_HMT = _HM.T.copy()
_HM2 = np.tile(np.eye(HEADS, dtype=np.float32), (1, 2))


def _silu(x):
    return x * (1.0 / (1.0 + jnp.exp(-x)))


def _lnorm(x, eps):
    mu = jnp.mean(x, axis=-1, keepdims=True)
    xc = x - mu
    var = jnp.mean(xc * xc, axis=-1, keepdims=True)
    return xc * lax.rsqrt(var + eps)


def _mm(a, b):
    return jnp.dot(a, b, preferred_element_type=F32)


def _node1_body(h, nte, pos16, ntW, ntb, q1W, q1b, qlg, qlb, q2W, q2b,
                osrc, odst):
    nt = _mm(_silu(nte[...]), ntW[...]) + ntb[...]
    nsh = nt[:, :NODE_DIM]
    nsc = nt[:, NODE_DIM:]
    hm = _lnorm(h[...], 1e-6) * (1.0 + nsc) + nsh
    qh = _silu(_lnorm(_mm(hm, q1W[...]) + q1b[...], 1e-5) * qlg[...] + qlb[...])
    q = _mm(qh, q2W[...]) + q2b[...]
    p16 = pos16[...]
    osrc[...] = jnp.concatenate([hm, p16], axis=1)
    odst[...] = jnp.concatenate([hm, q, p16], axis=1)


def _node1(h, nte, pos16, p):
    N = h.shape[0]
    B = 1000
    full = lambda a: pl.BlockSpec(jnp.shape(a), lambda i: (0,) * jnp.ndim(a))
    row = lambda c: pl.BlockSpec((B, c), lambda i: (i, 0))
    args = (h, nte, pos16,
            p['node_time_W'], p['node_time_b'].reshape(1, -1),
            p['q1_W'], p['q1_b'].reshape(1, -1),
            p['q_ln_g'].reshape(1, -1), p['q_ln_b'].reshape(1, -1),
            p['q2_W'], p['q2_b'].reshape(1, -1))
    in_specs = [row(128), row(128), row(16)] + [full(a) for a in args[3:]]
    return pl.pallas_call(
        _node1_body,
        grid=(N // B,),
        in_specs=in_specs,
        out_specs=[row(144), row(272)],
        out_shape=[jax.ShapeDtypeStruct((N, 144), F32),
                   jax.ShapeDtypeStruct((N, 272), F32)],
    )(*args)


def _sc_gather(tsrc, tdst, src2, dst2):
    E = src2.shape[0] * 128
    nchunks = E // 128
    nfull = nchunks // 32
    rem = nchunks - nfull * 32
    mesh = plsc.VectorSubcoreMesh(core_axis_name="c", subcore_axis_name="s")

    @functools.partial(
        pl.kernel,
        out_type=[jax.ShapeDtypeStruct((E, 144), F32),
                  jax.ShapeDtypeStruct((E, 272), F32)],
        mesh=mesh,
        scratch_types=[pltpu.VMEM((128,), jnp.int32),
                       pltpu.VMEM((128,), jnp.int32),
                       pltpu.VMEM((128, 144), F32),
                       pltpu.VMEM((128, 272), F32),
                       pltpu.SemaphoreType.DMA,
                       pltpu.SemaphoreType.DMA],
        compiler_params=pltpu.CompilerParams(use_tc_tiling_on_sc=False),
    )
    def k(ts, td, s2, d2, gs, gd, idx_s, idx_d, rs, rd, sem1, sem2):
        wid = lax.axis_index("s") * 2 + lax.axis_index("c")

        def chunk(c):
            base = c * 128
            pltpu.sync_copy(s2.at[c], idx_s)
            pltpu.sync_copy(d2.at[c], idx_d)
            c1 = pltpu.async_copy(ts.at[idx_s], rs, sem1)
            c2 = pltpu.async_copy(td.at[idx_d], rd, sem2)
            c1.wait()
            c2.wait()
            pltpu.sync_copy(rs, gs.at[pl.ds(base, 128)])
            pltpu.sync_copy(rd, gd.at[pl.ds(base, 128)])

        def body(j, carry):
            chunk(wid + 32 * j)
            return carry

        lax.fori_loop(0, nfull, body, 0)

        @pl.when(wid < rem)
        def _():
            chunk(wid + 32 * nfull)

    return k(tsrc, tdst, src2, dst2)


def _edge_body(gs, gd, eattr, ecl, ete,
               Wce, Wde, Wae, eb, etW, etb,
               ge, be, gi, bi, gj, bj,
               Ke, Ki, Kj, k1b, klg, klb, k2W, k2b,
               Ve, Vi, Vj, v1b, vlg, vlb, v2W, v2b,
               offs, Hm, HmT, Hm2,
               wout):
    hi = gs[:, :128]
    ps = gs[:, 128:144]
    hj = gd[:, :128]
    qd = gd[:, 128:256]
    pd = gd[:, 256:272]

    dp = ps - pd
    d2 = jnp.sum(dp * dp, axis=-1, keepdims=True)
    dist = jnp.sqrt(d2 + 1e-12)
    df = jnp.exp(_COEFF * (dist - offs[...]) ** 2)

    ea = _mm(ecl[...], Wce[...]) + _mm(df, Wde[...]) + _mm(eattr[...], Wae[...]) + eb[...]
    et = _mm(_silu(ete[...]), etW[...]) + etb[...]
    esh = et[:, :EDGE_DIM]
    esc = et[:, EDGE_DIM:]
    eam = _lnorm(ea, 1e-6) * (1.0 + esc) + esh

    D = 2 * NODE_DIM + EDGE_DIM
    s = (jnp.sum(eam, axis=-1, keepdims=True)
         + jnp.sum(hi, axis=-1, keepdims=True)
         + jnp.sum(hj, axis=-1, keepdims=True))
    ss = (jnp.sum(eam * eam, axis=-1, keepdims=True)
          + jnp.sum(hi * hi, axis=-1, keepdims=True)
          + jnp.sum(hj * hj, axis=-1, keepdims=True))
    mu = s / D
    var = ss / D - mu * mu
    inv = lax.rsqrt(var + 1e-5)
    xe = ((eam - mu) * inv) * ge[...] + be[...]
    xi = ((hi - mu) * inv) * gi[...] + bi[...]
    xj = ((hj - mu) * inv) * gj[...] + bj[...]

    k1 = _mm(xe, Ke[...]) + _mm(xi, Ki[...]) + _mm(xj, Kj[...]) + k1b[...]
    kk = _silu(_lnorm(k1, 1e-5) * klg[...] + klb[...])
    kv = _mm(kk, k2W[...]) + k2b[...]

    v1 = _mm(xe, Ve[...]) + _mm(xi, Vi[...]) + _mm(xj, Vj[...]) + v1b[...]
    vv = _silu(_lnorm(v1, 1e-5) * vlg[...] + vlb[...])
    v = _mm(vv, v2W[...]) + v2b[...]

    lg = _mm(qd * kv, Hm[...]) * 0.25
    ex = jnp.exp(lg)
    exb = _mm(ex, HmT[...])
    wout[...] = jnp.concatenate([exb * v, _mm(ex, Hm2[...])], axis=1)


def _edge(gsrc, gdst, edge_attr, clash_feat, ete, p):
    E = gsrc.shape[0]
    B = 640
    lng = p['lin_norm_g']
    lnb = p['lin_norm_b']
    r1 = lambda a: a.reshape(1, -1)
    args = (gsrc, gdst, edge_attr, clash_feat, ete,
            p['edge_emb_W'][:16], p['edge_emb_W'][16:80], p['edge_emb_W'][80:96],
            r1(p['edge_emb_b']), p['edge_time_W'], r1(p['edge_time_b']),
            r1(lng[:16]), r1(lnb[:16]), r1(lng[16:144]), r1(lnb[16:144]),
            r1(lng[144:272]), r1(lnb[144:272]),
            p['k1_W'][:16], p['k1_W'][16:144], p['k1_W'][144:272], r1(p['k1_b']),
            r1(p['k_ln_g']), r1(p['k_ln_b']), p['k2_W'], r1(p['k2_b']),
            p['v1_W'][:16], p['v1_W'][16:144], p['v1_W'][144:272], r1(p['v1_b']),
            r1(p['v_ln_g']), r1(p['v_ln_b']), p['v2_W'], r1(p['v2_b']),
            _OFFS.reshape(1, -1), _HM, _HMT, _HM2)
    row = lambda c: pl.BlockSpec((B, c), lambda i: (i, 0))
    full = lambda a: pl.BlockSpec(jnp.shape(a), lambda i: (0,) * jnp.ndim(a))
    in_specs = [row(144), row(272), row(16), row(16), row(128)] + \
               [full(a) for a in args[5:]]
    return pl.pallas_call(
        _edge_body,
        grid=(E // B,),
        in_specs=in_specs,
        out_specs=row(WROW),
        out_shape=jax.ShapeDtypeStruct((E, WROW), F32),
    )(*args)


def _sc_scatter(w, dst2, zrows, N):
    E = w.shape[0]
    nchunks = E // 128
    nfull = nchunks // 32
    rem = nchunks - nfull * 32
    rpt = N // 16
    mesh = plsc.VectorSubcoreMesh(core_axis_name="c", subcore_axis_name="s")

    @functools.partial(
        pl.kernel,
        out_type=jax.ShapeDtypeStruct((2 * N, WROW), F32),
        mesh=mesh,
        scratch_types=[pltpu.VMEM((128,), jnp.int32),
                       pltpu.VMEM((128, WROW), F32),
                       pltpu.VMEM_SHARED((N, WROW), F32)],
        compiler_params=pltpu.CompilerParams(use_tc_tiling_on_sc=False),
    )
    def k(wv, d2, z, out, idx_v, w_v, table):
        c = lax.axis_index("c")
        s = lax.axis_index("s")
        wid = s * 2 + c
        r0 = s * rpt
        pltpu.sync_copy(z, table.at[pl.ds(r0, rpt)])
        plsc.subcore_barrier()

        def chunk(ci):
            pltpu.sync_copy(d2.at[ci], idx_v)
            pltpu.sync_copy(wv.at[pl.ds(ci * 128, 128)], w_v)
            pltpu.sync_copy(w_v, table.at[idx_v], add=True)

        def body(j, carry):
            chunk(wid + 32 * j)
            return carry

        lax.fori_loop(0, nfull, body, 0)

        @pl.when(wid < rem)
        def _():
            chunk(wid + 32 * nfull)

        plsc.subcore_barrier()
        pltpu.sync_copy(table.at[pl.ds(r0, rpt)],
                        out.at[pl.ds(c * N + r0, rpt)])

    return k(w, dst2, zrows)


def _node2_body(p0, p1, h, o1W, o1b, o2W, o2b, f1W, f1b, f2W, f2b, HmT, hout):
    acc = p0[...] + p1[...]
    num = acc[:, :128]
    den = acc[:, 128:136]
    deb = _mm(den, HmT[...])
    out = num / (deb + 1e-16)
    o = _mm(_silu(_mm(out, o1W[...]) + o1b[...]), o2W[...]) + o2b[...]
    f = _mm(_silu(_mm(o, f1W[...]) + f1b[...]), f2W[...]) + f2b[...]
    hout[...] = h[...] + f


def _node2(part0, part1, h, p):
    N = h.shape[0]
    B = 1000
    r1 = lambda a: a.reshape(1, -1)
    args = (part0, part1, h,
            p['out1_W'], r1(p['out1_b']), p['out2_W'], r1(p['out2_b']),
            p['ff1_W'], r1(p['ff1_b']), p['ff2_W'], r1(p['ff2_b']), _HMT)
    row = lambda c: pl.BlockSpec((B, c), lambda i: (i, 0))
    full = lambda a: pl.BlockSpec(jnp.shape(a), lambda i: (0,) * jnp.ndim(a))
    in_specs = [row(144), row(144), row(128)] + [full(a) for a in args[3:]]
    return pl.pallas_call(
        _node2_body,
        grid=(N // B,),
        in_specs=in_specs,
        out_specs=row(128),
        out_shape=jax.ShapeDtypeStruct((N, 128), F32),
    )(*args)


def kernel(pos, h, edge_attr, clash_feat, edge_index, node_time_emb,
           edge_time_emb, params):
    N = h.shape[0]
    E = edge_attr.shape[0]
    src2 = edge_index[0].reshape(E // 128, 128)
    dst2 = edge_index[1].reshape(E // 128, 128)
    pos16 = jnp.pad(pos, ((0, 0), (0, 13)))

    tsrc, tdst = _node1(h, node_time_emb, pos16, params)
    gsrc, gdst = _sc_gather(tsrc, tdst, src2, dst2)
    w = _edge(gsrc, gdst, edge_attr, clash_feat, edge_time_emb, params)
    zrows = jnp.zeros((N // 16, WROW), F32)
    parts = _sc_scatter(w, dst2, zrows, N)
    h_node = _node2(parts[:N], parts[N:], h, params)
    return (h_node, pos)


# speedup vs baseline: 13.5077x; 1.2628x over previous
import functools

import jax
import jax.numpy as jnp
import numpy as np
from jax import lax
from jax.experimental import pallas as pl
from jax.experimental.pallas import tpu as pltpu
from jax.experimental.pallas import tpu_sc as plsc

F32 = jnp.float32

NODE_DIM = 128
EDGE_DIM = 16
DIST_DIM = 64
HEADS = 8
HEAD_C = 16

_OFFS = np.linspace(0.0, 15.0, DIST_DIM).astype(np.float32)
_COEFF = float(-0.5 / (_OFFS[1] - _OFFS[0]) ** 2)
_HM = np.kron(np.eye(HEADS, dtype=np.float32), np.ones((HEAD_C, 1), np.float32))
_HMT = _HM.T.copy()
_HM2 = np.tile(np.eye(HEADS, dtype=np.float32), (1, 2))


def _silu(x):
    return x * (1.0 / (1.0 + jnp.exp(-x)))


def _lnorm(x, eps):
    mu = jnp.mean(x, axis=-1, keepdims=True)
    xc = x - mu
    var = jnp.mean(xc * xc, axis=-1, keepdims=True)
    return xc * lax.rsqrt(var + eps)


def _mm(a, b):
    return jnp.dot(a, b, preferred_element_type=F32)


def _node1_body(h, nte, ntW, ntb, q1W, q1b, qlg, qlb, q2W, q2b,
                ohm, oq):
    nt = _mm(_silu(nte[...]), ntW[...]) + ntb[...]
    nsh = nt[:, :NODE_DIM]
    nsc = nt[:, NODE_DIM:]
    hm = _lnorm(h[...], 1e-6) * (1.0 + nsc) + nsh
    qh = _silu(_lnorm(_mm(hm, q1W[...]) + q1b[...], 1e-5) * qlg[...] + qlb[...])
    oq[...] = _mm(qh, q2W[...]) + q2b[...]
    ohm[...] = hm


def _node1(h, nte, p):
    N = h.shape[0]
    B = 1000
    full = lambda a: pl.BlockSpec(jnp.shape(a), lambda i: (0,) * jnp.ndim(a))
    row = lambda c: pl.BlockSpec((B, c), lambda i: (i, 0))
    args = (h, nte,
            p['node_time_W'], p['node_time_b'].reshape(1, -1),
            p['q1_W'], p['q1_b'].reshape(1, -1),
            p['q_ln_g'].reshape(1, -1), p['q_ln_b'].reshape(1, -1),
            p['q2_W'], p['q2_b'].reshape(1, -1))
    in_specs = [row(128), row(128)] + [full(a) for a in args[2:]]
    return pl.pallas_call(
        _node1_body,
        grid=(N // B,),
        in_specs=in_specs,
        out_specs=[row(128), row(128)],
        out_shape=[jax.ShapeDtypeStruct((N, 128), F32),
                   jax.ShapeDtypeStruct((N, 128), F32)],
    )(*args)


def _sc_gather(t_hm, t_q, t_pos, src2, dst2):
    E = src2.shape[0] * 128
    nchunks = E // 128
    nfull = nchunks // 32
    rem = nchunks - nfull * 32
    mesh = plsc.VectorSubcoreMesh(core_axis_name="c", subcore_axis_name="s")

    @functools.partial(
        pl.kernel,
        out_type=[jax.ShapeDtypeStruct((E, 128), F32),
                  jax.ShapeDtypeStruct((E, 16), F32),
                  jax.ShapeDtypeStruct((E, 128), F32),
                  jax.ShapeDtypeStruct((E, 128), F32),
                  jax.ShapeDtypeStruct((E, 16), F32)],
        mesh=mesh,
        scratch_types=[pltpu.VMEM((128,), jnp.int32),
                       pltpu.VMEM((128,), jnp.int32),
                       pltpu.VMEM((128, 128), F32),
                       pltpu.VMEM((128, 16), F32),
                       pltpu.VMEM((128, 128), F32),
                       pltpu.VMEM((128, 128), F32),
                       pltpu.VMEM((128, 16), F32),
                       pltpu.SemaphoreType.DMA,
                       pltpu.SemaphoreType.DMA,
                       pltpu.SemaphoreType.DMA,
                       pltpu.SemaphoreType.DMA,
                       pltpu.SemaphoreType.DMA],
        compiler_params=pltpu.CompilerParams(use_tc_tiling_on_sc=False),
    )
    def k(thm, tq, tpos, s2, d2, ghs, gps, ghd, gqd, gpd,
          idx_s, idx_d, r_hs, r_ps, r_hd, r_qd, r_pd,
          m1, m2, m3, m4, m5):
        wid = lax.axis_index("s") * 2 + lax.axis_index("c")

        def chunk(c):
            base = c * 128
            pltpu.sync_copy(s2.at[c], idx_s)
            pltpu.sync_copy(d2.at[c], idx_d)
            c1 = pltpu.async_copy(thm.at[idx_s], r_hs, m1)
            c2 = pltpu.async_copy(tpos.at[idx_s], r_ps, m2)
            c3 = pltpu.async_copy(thm.at[idx_d], r_hd, m3)
            c4 = pltpu.async_copy(tq.at[idx_d], r_qd, m4)
            c5 = pltpu.async_copy(tpos.at[idx_d], r_pd, m5)
            c1.wait(); c2.wait(); c3.wait(); c4.wait(); c5.wait()
            pltpu.sync_copy(r_hs, ghs.at[pl.ds(base, 128)])
            pltpu.sync_copy(r_ps, gps.at[pl.ds(base, 128)])
            pltpu.sync_copy(r_hd, ghd.at[pl.ds(base, 128)])
            pltpu.sync_copy(r_qd, gqd.at[pl.ds(base, 128)])
            pltpu.sync_copy(r_pd, gpd.at[pl.ds(base, 128)])

        def body(j, carry):
            chunk(wid + 32 * j)
            return carry

        lax.fori_loop(0, nfull, body, 0)

        @pl.when(wid < rem)
        def _():
            chunk(wid + 32 * nfull)

    return k(t_hm, t_q, t_pos, src2, dst2)


def _edge_body(ghs, gps, ghd, gqd, gpd, eattr, ecl, ete,
               Wce, Wde, Wae, eb, etW, etb,
               lng, lnb, Wkv, kvb,
               klg, klb, k2W, k2b,
               vlg, vlb, v2W, v2b,
               offs, Hm, HmT, Hm2,
               w1out, w2out):
    hi = ghs[...]
    hj = ghd[...]
    qd = gqd[...]

    dp = gps[...] - gpd[...]
    d2 = jnp.sum(dp * dp, axis=-1, keepdims=True)
    dist = jnp.sqrt(d2 + 1e-12)
    df = jnp.exp(_COEFF * (dist - offs[...]) ** 2)

    ea = _mm(ecl[...], Wce[...]) + _mm(df, Wde[...]) + _mm(eattr[...], Wae[...]) + eb[...]
    et = _mm(_silu(ete[...]), etW[...]) + etb[...]
    esh = et[:, :EDGE_DIM]
    esc = et[:, EDGE_DIM:]
    eam = _lnorm(ea, 1e-6) * (1.0 + esc) + esh

    x = jnp.concatenate([eam, hi, hj], axis=1)
    xf = _lnorm(x, 1e-5) * lng[...] + lnb[...]
    kv1 = _mm(xf, Wkv[...]) + kvb[...]
    k1 = kv1[:, :NODE_DIM]
    v1 = kv1[:, NODE_DIM:]

    kk = _silu(_lnorm(k1, 1e-5) * klg[...] + klb[...])
    kv = _mm(kk, k2W[...]) + k2b[...]

    vv = _silu(_lnorm(v1, 1e-5) * vlg[...] + vlb[...])
    v = _mm(vv, v2W[...]) + v2b[...]

    lg = _mm(qd * kv, Hm[...]) * 0.25
    ex = jnp.exp(lg)
    exb = _mm(ex, HmT[...])
    w1out[...] = exb * v
    w2out[...] = _mm(ex, Hm2[...])


def _edge(ghs, gps, ghd, gqd, gpd, edge_attr, clash_feat, ete, p):
    E = ghs.shape[0]
    B = 640
    r1 = lambda a: a.reshape(1, -1)
    Wkv = jnp.concatenate([p['k1_W'], p['v1_W']], axis=1)
    kvb = jnp.concatenate([p['k1_b'], p['v1_b']]).reshape(1, -1)
    args = (ghs, gps, ghd, gqd, gpd, edge_attr, clash_feat, ete,
            p['edge_emb_W'][:16], p['edge_emb_W'][16:80], p['edge_emb_W'][80:96],
            r1(p['edge_emb_b']), p['edge_time_W'], r1(p['edge_time_b']),
            r1(p['lin_norm_g']), r1(p['lin_norm_b']), Wkv, kvb,
            r1(p['k_ln_g']), r1(p['k_ln_b']), p['k2_W'], r1(p['k2_b']),
            r1(p['v_ln_g']), r1(p['v_ln_b']), p['v2_W'], r1(p['v2_b']),
            _OFFS.reshape(1, -1), _HM, _HMT, _HM2)
    row = lambda c: pl.BlockSpec((B, c), lambda i: (i, 0))
    full = lambda a: pl.BlockSpec(jnp.shape(a), lambda i: (0,) * jnp.ndim(a))
    in_specs = [row(128), row(16), row(128), row(128), row(16),
                row(16), row(16), row(128)] + [full(a) for a in args[8:]]
    return pl.pallas_call(
        _edge_body,
        grid=(E // B,),
        in_specs=in_specs,
        out_specs=[row(128), row(16)],
        out_shape=[jax.ShapeDtypeStruct((E, 128), F32),
                   jax.ShapeDtypeStruct((E, 16), F32)],
    )(*args)


def _sc_scatter(w1, w2, dst2, z1, z2, N):
    E = w1.shape[0]
    nchunks = E // 128
    nfull = nchunks // 32
    rem = nchunks - nfull * 32
    rpt = N // 16
    mesh = plsc.VectorSubcoreMesh(core_axis_name="c", subcore_axis_name="s")

    @functools.partial(
        pl.kernel,
        out_type=[jax.ShapeDtypeStruct((2 * N, 128), F32),
                  jax.ShapeDtypeStruct((2 * N, 16), F32)],
        mesh=mesh,
        scratch_types=[pltpu.VMEM((128,), jnp.int32),
                       pltpu.VMEM((128, 128), F32),
                       pltpu.VMEM((128, 16), F32),
                       pltpu.VMEM_SHARED((N, 128), F32),
                       pltpu.VMEM_SHARED((N, 16), F32)],
        compiler_params=pltpu.CompilerParams(use_tc_tiling_on_sc=False),
    )
    def k(wv1, wv2, d2, zz1, zz2, outA, outB, idx_v, w1v, w2v, tabA, tabB):
        c = lax.axis_index("c")
        s = lax.axis_index("s")
        wid = s * 2 + c
        r0 = s * rpt
        pltpu.sync_copy(zz1, tabA.at[pl.ds(r0, rpt)])
        pltpu.sync_copy(zz2, tabB.at[pl.ds(r0, rpt)])
        plsc.subcore_barrier()

        def chunk(ci):
            pltpu.sync_copy(d2.at[ci], idx_v)
            pltpu.sync_copy(wv1.at[pl.ds(ci * 128, 128)], w1v)
            pltpu.sync_copy(wv2.at[pl.ds(ci * 128, 128)], w2v)
            pltpu.sync_copy(w1v, tabA.at[idx_v], add=True)
            pltpu.sync_copy(w2v, tabB.at[idx_v], add=True)

        def body(j, carry):
            chunk(wid + 32 * j)
            return carry

        lax.fori_loop(0, nfull, body, 0)

        @pl.when(wid < rem)
        def _():
            chunk(wid + 32 * nfull)

        plsc.subcore_barrier()
        pltpu.sync_copy(tabA.at[pl.ds(r0, rpt)], outA.at[pl.ds(c * N + r0, rpt)])
        pltpu.sync_copy(tabB.at[pl.ds(r0, rpt)], outB.at[pl.ds(c * N + r0, rpt)])

    return k(w1, w2, dst2, z1, z2)


def _node2_body(pA0, pA1, pB0, pB1, h, o1W, o1b, o2W, o2b,
                f1W, f1b, f2W, f2b, HmT, hout):
    num = pA0[...] + pA1[...]
    den = pB0[:, :HEADS] + pB1[:, :HEADS]
    deb = _mm(den, HmT[...])
    out = num / (deb + 1e-16)
    o = _mm(_silu(_mm(out, o1W[...]) + o1b[...]), o2W[...]) + o2b[...]
    f = _mm(_silu(_mm(o, f1W[...]) + f1b[...]), f2W[...]) + f2b[...]
    hout[...] = h[...] + f


def _node2(pA0, pA1, pB0, pB1, h, p):
    N = h.shape[0]
    B = 1000
    r1 = lambda a: a.reshape(1, -1)
    args = (pA0, pA1, pB0, pB1, h,
            p['out1_W'], r1(p['out1_b']), p['out2_W'], r1(p['out2_b']),
            p['ff1_W'], r1(p['ff1_b']), p['ff2_W'], r1(p['ff2_b']), _HMT)
    row = lambda c: pl.BlockSpec((B, c), lambda i: (i, 0))
    full = lambda a: pl.BlockSpec(jnp.shape(a), lambda i: (0,) * jnp.ndim(a))
    in_specs = [row(128), row(128), row(16), row(16), row(128)] + \
               [full(a) for a in args[5:]]
    return pl.pallas_call(
        _node2_body,
        grid=(N // B,),
        in_specs=in_specs,
        out_specs=row(128),
        out_shape=jax.ShapeDtypeStruct((N, 128), F32),
    )(*args)


def kernel(pos, h, edge_attr, clash_feat, edge_index, node_time_emb,
           edge_time_emb, params):
    N = h.shape[0]
    E = edge_attr.shape[0]
    src2 = edge_index[0].reshape(E // 128, 128)
    dst2 = edge_index[1].reshape(E // 128, 128)
    pos16 = jnp.pad(pos, ((0, 0), (0, 13)))

    t_hm, t_q = _node1(h, node_time_emb, params)
    ghs, gps, ghd, gqd, gpd = _sc_gather(t_hm, t_q, pos16, src2, dst2)
    w1, w2 = _edge(ghs, gps, ghd, gqd, gpd, edge_attr, clash_feat,
                   edge_time_emb, params)
    z1 = jnp.zeros((N // 16, 128), F32)
    z2 = jnp.zeros((N // 16, 16), F32)
    partsA, partsB = _sc_scatter(w1, w2, dst2, z1, z2, N)
    h_node = _node2(partsA[:N], partsA[N:], partsB[:N], partsB[N:], h, params)
    return (h_node, pos)


# speedup vs baseline: 14.9045x; 1.1034x over previous
import functools

import jax
import jax.numpy as jnp
import numpy as np
from jax import lax
from jax.experimental import pallas as pl
from jax.experimental.pallas import tpu as pltpu
from jax.experimental.pallas import tpu_sc as plsc

F32 = jnp.float32

NODE_DIM = 128
EDGE_DIM = 16
DIST_DIM = 64
HEADS = 8
HEAD_C = 16

_OFFS = np.linspace(0.0, 15.0, DIST_DIM).astype(np.float32)
_COEFF = float(-0.5 / (_OFFS[1] - _OFFS[0]) ** 2)
_HM = np.kron(np.eye(HEADS, dtype=np.float32), np.ones((HEAD_C, 1), np.float32))
_HMT = _HM.T.copy()
_HM2 = np.tile(np.eye(HEADS, dtype=np.float32), (1, 2))


def _silu(x):
    return x * (1.0 / (1.0 + jnp.exp(-x)))


def _lnorm(x, eps):
    mu = jnp.mean(x, axis=-1, keepdims=True)
    xc = x - mu
    var = jnp.mean(xc * xc, axis=-1, keepdims=True)
    return xc * lax.rsqrt(var + eps)


def _mm(a, b):
    return jnp.dot(a, b, preferred_element_type=F32)


def _mmb(a, b):
    return jnp.dot(a.astype(jnp.bfloat16), b, preferred_element_type=F32)


def _node1_body(h, nte, ntW, ntb, q1W, q1b, qlg, qlb, q2W, q2b,
                ohm, oq):
    nt = _mm(_silu(nte[...]), ntW[...]) + ntb[...]
    nsh = nt[:, :NODE_DIM]
    nsc = nt[:, NODE_DIM:]
    hm = _lnorm(h[...], 1e-6) * (1.0 + nsc) + nsh
    qh = _silu(_lnorm(_mm(hm, q1W[...]) + q1b[...], 1e-5) * qlg[...] + qlb[...])
    oq[...] = _mm(qh, q2W[...]) + q2b[...]
    ohm[...] = hm


def _node1(h, nte, p):
    N = h.shape[0]
    B = 1000
    full = lambda a: pl.BlockSpec(jnp.shape(a), lambda i: (0,) * jnp.ndim(a))
    row = lambda c: pl.BlockSpec((B, c), lambda i: (i, 0))
    args = (h, nte,
            p['node_time_W'], p['node_time_b'].reshape(1, -1),
            p['q1_W'], p['q1_b'].reshape(1, -1),
            p['q_ln_g'].reshape(1, -1), p['q_ln_b'].reshape(1, -1),
            p['q2_W'], p['q2_b'].reshape(1, -1))
    in_specs = [row(128), row(128)] + [full(a) for a in args[2:]]
    return pl.pallas_call(
        _node1_body,
        grid=(N // B,),
        in_specs=in_specs,
        out_specs=[row(128), row(128)],
        out_shape=[jax.ShapeDtypeStruct((N, 128), F32),
                   jax.ShapeDtypeStruct((N, 128), F32)],
    )(*args)


def _sc_gather(t_hm, t_q, t_pos, src2, dst2):
    E = src2.shape[0] * 128
    nchunks = E // 128
    nfull = nchunks // 32
    rem = nchunks - nfull * 32
    mesh = plsc.VectorSubcoreMesh(core_axis_name="c", subcore_axis_name="s")

    @functools.partial(
        pl.kernel,
        out_type=[jax.ShapeDtypeStruct((E, 128), F32),
                  jax.ShapeDtypeStruct((E, 16), F32),
                  jax.ShapeDtypeStruct((E, 128), F32),
                  jax.ShapeDtypeStruct((E, 128), F32),
                  jax.ShapeDtypeStruct((E, 16), F32)],
        mesh=mesh,
        scratch_types=[pltpu.VMEM((128,), jnp.int32),
                       pltpu.VMEM((128,), jnp.int32),
                       pltpu.VMEM((128, 128), F32),
                       pltpu.VMEM((128, 16), F32),
                       pltpu.VMEM((128, 128), F32),
                       pltpu.VMEM((128, 128), F32),
                       pltpu.VMEM((128, 16), F32),
                       pltpu.SemaphoreType.DMA,
                       pltpu.SemaphoreType.DMA,
                       pltpu.SemaphoreType.DMA,
                       pltpu.SemaphoreType.DMA,
                       pltpu.SemaphoreType.DMA],
        compiler_params=pltpu.CompilerParams(use_tc_tiling_on_sc=False),
    )
    def k(thm, tq, tpos, s2, d2, ghs, gps, ghd, gqd, gpd,
          idx_s, idx_d, r_hs, r_ps, r_hd, r_qd, r_pd,
          m1, m2, m3, m4, m5):
        wid = lax.axis_index("s") * 2 + lax.axis_index("c")

        def chunk(c):
            base = c * 128
            pltpu.sync_copy(s2.at[c], idx_s)
            pltpu.sync_copy(d2.at[c], idx_d)
            c1 = pltpu.async_copy(thm.at[idx_s], r_hs, m1)
            c2 = pltpu.async_copy(tpos.at[idx_s], r_ps, m2)
            c3 = pltpu.async_copy(thm.at[idx_d], r_hd, m3)
            c4 = pltpu.async_copy(tq.at[idx_d], r_qd, m4)
            c5 = pltpu.async_copy(tpos.at[idx_d], r_pd, m5)
            c1.wait(); c2.wait(); c3.wait(); c4.wait(); c5.wait()
            pltpu.sync_copy(r_hs, ghs.at[pl.ds(base, 128)])
            pltpu.sync_copy(r_ps, gps.at[pl.ds(base, 128)])
            pltpu.sync_copy(r_hd, ghd.at[pl.ds(base, 128)])
            pltpu.sync_copy(r_qd, gqd.at[pl.ds(base, 128)])
            pltpu.sync_copy(r_pd, gpd.at[pl.ds(base, 128)])

        def body(j, carry):
            chunk(wid + 32 * j)
            return carry

        lax.fori_loop(0, nfull, body, 0)

        @pl.when(wid < rem)
        def _():
            chunk(wid + 32 * nfull)

    return k(t_hm, t_q, t_pos, src2, dst2)


def _edge_body(ghs, gps, ghd, gqd, gpd, eattr, ecl, ete,
               Wce, Wde, Wae, eb, etW, etb,
               lng, lnb, Wkv, kvb,
               klg, klb, k2W, k2b,
               vlg, vlb, v2W, v2b,
               offs, Hm, HmT, Hm2,
               w1out, w2out):
    hi = ghs[...]
    hj = ghd[...]
    qd = gqd[...]

    dp = gps[...] - gpd[...]
    d2 = jnp.sum(dp * dp, axis=-1, keepdims=True)
    dist = jnp.sqrt(d2 + 1e-12)
    df = jnp.exp(_COEFF * (dist - offs[...]) ** 2)

    ea = _mm(ecl[...], Wce[...]) + _mm(df, Wde[...]) + _mm(eattr[...], Wae[...]) + eb[...]
    et = _mm(_silu(ete[...]), etW[...]) + etb[...]
    esh = et[:, :EDGE_DIM]
    esc = et[:, EDGE_DIM:]
    eam = _lnorm(ea, 1e-6) * (1.0 + esc) + esh

    x = jnp.concatenate([eam, hi, hj], axis=1)
    xf = _lnorm(x, 1e-5) * lng[...] + lnb[...]
    kv1 = _mmb(xf, Wkv[...]) + kvb[...]
    k1 = kv1[:, :NODE_DIM]
    v1 = kv1[:, NODE_DIM:]

    kk = _silu(_lnorm(k1, 1e-5) * klg[...] + klb[...])
    kv = _mmb(kk, k2W[...]) + k2b[...]

    vv = _silu(_lnorm(v1, 1e-5) * vlg[...] + vlb[...])
    v = _mmb(vv, v2W[...]) + v2b[...]

    lg = _mm(qd * kv, Hm[...]) * 0.25
    ex = jnp.exp(lg)
    exb = _mm(ex, HmT[...])
    w1out[...] = exb * v
    w2out[...] = _mm(ex, Hm2[...])


def _edge(ghs, gps, ghd, gqd, gpd, edge_attr, clash_feat, ete, p):
    E = ghs.shape[0]
    B = 1280
    r1 = lambda a: a.reshape(1, -1)
    Wkv = jnp.concatenate([p['k1_W'], p['v1_W']], axis=1).astype(jnp.bfloat16)
    kvb = jnp.concatenate([p['k1_b'], p['v1_b']]).reshape(1, -1)
    args = (ghs, gps, ghd, gqd, gpd, edge_attr, clash_feat, ete,
            p['edge_emb_W'][:16], p['edge_emb_W'][16:80], p['edge_emb_W'][80:96],
            r1(p['edge_emb_b']), p['edge_time_W'], r1(p['edge_time_b']),
            r1(p['lin_norm_g']), r1(p['lin_norm_b']), Wkv, kvb,
            r1(p['k_ln_g']), r1(p['k_ln_b']),
            p['k2_W'].astype(jnp.bfloat16), r1(p['k2_b']),
            r1(p['v_ln_g']), r1(p['v_ln_b']),
            p['v2_W'].astype(jnp.bfloat16), r1(p['v2_b']),
            _OFFS.reshape(1, -1), _HM, _HMT, _HM2)
    row = lambda c: pl.BlockSpec((B, c), lambda i: (i, 0))
    full = lambda a: pl.BlockSpec(jnp.shape(a), lambda i: (0,) * jnp.ndim(a))
    in_specs = [row(128), row(16), row(128), row(128), row(16),
                row(16), row(16), row(128)] + [full(a) for a in args[8:]]
    return pl.pallas_call(
        _edge_body,
        grid=(E // B,),
        in_specs=in_specs,
        out_specs=[row(128), row(16)],
        out_shape=[jax.ShapeDtypeStruct((E, 128), F32),
                   jax.ShapeDtypeStruct((E, 16), F32)],
    )(*args)


def _sc_scatter(w1, w2, dst2, z1, z2, N):
    E = w1.shape[0]
    nchunks = E // 128
    nfull = nchunks // 32
    rem = nchunks - nfull * 32
    rpt = N // 16
    mesh = plsc.VectorSubcoreMesh(core_axis_name="c", subcore_axis_name="s")

    @functools.partial(
        pl.kernel,
        out_type=[jax.ShapeDtypeStruct((2 * N, 128), F32),
                  jax.ShapeDtypeStruct((2 * N, 16), F32)],
        mesh=mesh,
        scratch_types=[pltpu.VMEM((128,), jnp.int32),
                       pltpu.VMEM((128, 128), F32),
                       pltpu.VMEM((128, 16), F32),
                       pltpu.VMEM_SHARED((N, 128), F32),
                       pltpu.VMEM_SHARED((N, 16), F32)],
        compiler_params=pltpu.CompilerParams(use_tc_tiling_on_sc=False),
    )
    def k(wv1, wv2, d2, zz1, zz2, outA, outB, idx_v, w1v, w2v, tabA, tabB):
        c = lax.axis_index("c")
        s = lax.axis_index("s")
        wid = s * 2 + c
        r0 = s * rpt
        pltpu.sync_copy(zz1, tabA.at[pl.ds(r0, rpt)])
        pltpu.sync_copy(zz2, tabB.at[pl.ds(r0, rpt)])
        plsc.subcore_barrier()

        def chunk(ci):
            pltpu.sync_copy(d2.at[ci], idx_v)
            pltpu.sync_copy(wv1.at[pl.ds(ci * 128, 128)], w1v)
            pltpu.sync_copy(wv2.at[pl.ds(ci * 128, 128)], w2v)
            pltpu.sync_copy(w1v, tabA.at[idx_v], add=True)
            pltpu.sync_copy(w2v, tabB.at[idx_v], add=True)

        def body(j, carry):
            chunk(wid + 32 * j)
            return carry

        lax.fori_loop(0, nfull, body, 0)

        @pl.when(wid < rem)
        def _():
            chunk(wid + 32 * nfull)

        plsc.subcore_barrier()
        pltpu.sync_copy(tabA.at[pl.ds(r0, rpt)], outA.at[pl.ds(c * N + r0, rpt)])
        pltpu.sync_copy(tabB.at[pl.ds(r0, rpt)], outB.at[pl.ds(c * N + r0, rpt)])

    return k(w1, w2, dst2, z1, z2)


def _node2_body(pA0, pA1, pB0, pB1, h, o1W, o1b, o2W, o2b,
                f1W, f1b, f2W, f2b, HmT, hout):
    num = pA0[...] + pA1[...]
    den = pB0[:, :HEADS] + pB1[:, :HEADS]
    deb = _mm(den, HmT[...])
    out = num / (deb + 1e-16)
    o = _mm(_silu(_mm(out, o1W[...]) + o1b[...]), o2W[...]) + o2b[...]
    f = _mm(_silu(_mm(o, f1W[...]) + f1b[...]), f2W[...]) + f2b[...]
    hout[...] = h[...] + f


def _node2(pA0, pA1, pB0, pB1, h, p):
    N = h.shape[0]
    B = 1000
    r1 = lambda a: a.reshape(1, -1)
    args = (pA0, pA1, pB0, pB1, h,
            p['out1_W'], r1(p['out1_b']), p['out2_W'], r1(p['out2_b']),
            p['ff1_W'], r1(p['ff1_b']), p['ff2_W'], r1(p['ff2_b']), _HMT)
    row = lambda c: pl.BlockSpec((B, c), lambda i: (i, 0))
    full = lambda a: pl.BlockSpec(jnp.shape(a), lambda i: (0,) * jnp.ndim(a))
    in_specs = [row(128), row(128), row(16), row(16), row(128)] + \
               [full(a) for a in args[5:]]
    return pl.pallas_call(
        _node2_body,
        grid=(N // B,),
        in_specs=in_specs,
        out_specs=row(128),
        out_shape=jax.ShapeDtypeStruct((N, 128), F32),
    )(*args)


def kernel(pos, h, edge_attr, clash_feat, edge_index, node_time_emb,
           edge_time_emb, params):
    N = h.shape[0]
    E = edge_attr.shape[0]
    src2 = edge_index[0].reshape(E // 128, 128)
    dst2 = edge_index[1].reshape(E // 128, 128)
    pos16 = jnp.pad(pos, ((0, 0), (0, 13)))

    t_hm, t_q = _node1(h, node_time_emb, params)
    ghs, gps, ghd, gqd, gpd = _sc_gather(t_hm, t_q, pos16, src2, dst2)
    w1, w2 = _edge(ghs, gps, ghd, gqd, gpd, edge_attr, clash_feat,
                   edge_time_emb, params)
    z1 = jnp.zeros((N // 16, 128), F32)
    z2 = jnp.zeros((N // 16, 16), F32)
    partsA, partsB = _sc_scatter(w1, w2, dst2, z1, z2, N)
    h_node = _node2(partsA[:N], partsA[N:], partsB[:N], partsB[N:], h, params)
    return (h_node, pos)


# speedup vs baseline: 18.9696x; 1.2727x over previous
import functools

import jax
import jax.numpy as jnp
import numpy as np
from jax import lax
from jax.experimental import pallas as pl
from jax.experimental.pallas import tpu as pltpu
from jax.experimental.pallas import tpu_sc as plsc

F32 = jnp.float32

NODE_DIM = 128
EDGE_DIM = 16
DIST_DIM = 64
HEADS = 8
HEAD_C = 16

_OFFS = np.linspace(0.0, 15.0, DIST_DIM).astype(np.float32)
_COEFF = float(-0.5 / (_OFFS[1] - _OFFS[0]) ** 2)
_HM = np.kron(np.eye(HEADS, dtype=np.float32), np.ones((HEAD_C, 1), np.float32))
_HMT = _HM.T.copy()
_HM2 = np.tile(np.eye(HEADS, dtype=np.float32), (1, 2))


def _silu(x):
    return x * (1.0 / (1.0 + jnp.exp(-x)))


def _lnorm(x, eps):
    mu = jnp.mean(x, axis=-1, keepdims=True)
    xc = x - mu
    var = jnp.mean(xc * xc, axis=-1, keepdims=True)
    return xc * lax.rsqrt(var + eps)


def _mm(a, b):
    return jnp.dot(a, b, preferred_element_type=F32)


def _mmb(a, b):
    return jnp.dot(a.astype(jnp.bfloat16), b, preferred_element_type=F32)


def _node1_body(h, nte, pos16, u3, u4, ntW, ntb, q1W, q1b, qlg, qlb, q2W, q2b,
                ohm, oq, opx):
    nt = _mm(_silu(nte[...]), ntW[...]) + ntb[...]
    nsh = nt[:, :NODE_DIM]
    nsc = nt[:, NODE_DIM:]
    hm = _lnorm(h[...], 1e-6) * (1.0 + nsc) + nsh
    qh = _silu(_lnorm(_mm(hm, q1W[...]) + q1b[...], 1e-5) * qlg[...] + qlb[...])
    oq[...] = _mm(qh, q2W[...]) + q2b[...]
    ohm[...] = hm
    s1 = jnp.sum(hm, axis=-1, keepdims=True)
    s2 = jnp.sum(hm * hm, axis=-1, keepdims=True)
    opx[...] = pos16[...] + s1 * u3[...] + s2 * u4[...]


def _node1(h, nte, pos16, p):
    N = h.shape[0]
    B = 1000
    u3 = np.zeros((1, 16), np.float32); u3[0, 3] = 1.0
    u4 = np.zeros((1, 16), np.float32); u4[0, 4] = 1.0
    full = lambda a: pl.BlockSpec(jnp.shape(a), lambda i: (0,) * jnp.ndim(a))
    row = lambda c: pl.BlockSpec((B, c), lambda i: (i, 0))
    args = (h, nte, pos16, u3, u4,
            p['node_time_W'], p['node_time_b'].reshape(1, -1),
            p['q1_W'], p['q1_b'].reshape(1, -1),
            p['q_ln_g'].reshape(1, -1), p['q_ln_b'].reshape(1, -1),
            p['q2_W'], p['q2_b'].reshape(1, -1))
    in_specs = [row(128), row(128), row(16)] + [full(a) for a in args[3:]]
    return pl.pallas_call(
        _node1_body,
        grid=(N // B,),
        in_specs=in_specs,
        out_specs=[row(128), row(128), row(16)],
        out_shape=[jax.ShapeDtypeStruct((N, 128), F32),
                   jax.ShapeDtypeStruct((N, 128), F32),
                   jax.ShapeDtypeStruct((N, 16), F32)],
    )(*args)


def _sc_gather(t_hm, t_q, t_pos, src2, dst2):
    E = src2.shape[0] * 128
    nchunks = E // 128
    nfull = nchunks // 32
    rem = nchunks - nfull * 32
    mesh = plsc.VectorSubcoreMesh(core_axis_name="c", subcore_axis_name="s")

    @functools.partial(
        pl.kernel,
        out_type=[jax.ShapeDtypeStruct((E, 128), F32),
                  jax.ShapeDtypeStruct((E, 16), F32),
                  jax.ShapeDtypeStruct((E, 128), F32),
                  jax.ShapeDtypeStruct((E, 128), F32),
                  jax.ShapeDtypeStruct((E, 16), F32)],
        mesh=mesh,
        scratch_types=[pltpu.VMEM((128,), jnp.int32),
                       pltpu.VMEM((128,), jnp.int32),
                       pltpu.VMEM((128, 128), F32),
                       pltpu.VMEM((128, 16), F32),
                       pltpu.VMEM((128, 128), F32),
                       pltpu.VMEM((128, 128), F32),
                       pltpu.VMEM((128, 16), F32),
                       pltpu.SemaphoreType.DMA,
                       pltpu.SemaphoreType.DMA,
                       pltpu.SemaphoreType.DMA,
                       pltpu.SemaphoreType.DMA,
                       pltpu.SemaphoreType.DMA],
        compiler_params=pltpu.CompilerParams(use_tc_tiling_on_sc=False),
    )
    def k(thm, tq, tpos, s2, d2, ghs, gps, ghd, gqd, gpd,
          idx_s, idx_d, r_hs, r_ps, r_hd, r_qd, r_pd,
          m1, m2, m3, m4, m5):
        wid = lax.axis_index("s") * 2 + lax.axis_index("c")

        def chunk(c):
            base = c * 128
            pltpu.sync_copy(s2.at[c], idx_s)
            pltpu.sync_copy(d2.at[c], idx_d)
            c1 = pltpu.async_copy(thm.at[idx_s], r_hs, m1)
            c2 = pltpu.async_copy(tpos.at[idx_s], r_ps, m2)
            c3 = pltpu.async_copy(thm.at[idx_d], r_hd, m3)
            c4 = pltpu.async_copy(tq.at[idx_d], r_qd, m4)
            c5 = pltpu.async_copy(tpos.at[idx_d], r_pd, m5)
            c1.wait(); c2.wait(); c3.wait(); c4.wait(); c5.wait()
            pltpu.sync_copy(r_hs, ghs.at[pl.ds(base, 128)])
            pltpu.sync_copy(r_ps, gps.at[pl.ds(base, 128)])
            pltpu.sync_copy(r_hd, ghd.at[pl.ds(base, 128)])
            pltpu.sync_copy(r_qd, gqd.at[pl.ds(base, 128)])
            pltpu.sync_copy(r_pd, gpd.at[pl.ds(base, 128)])

        def body(j, carry):
            chunk(wid + 32 * j)
            return carry

        lax.fori_loop(0, nfull, body, 0)

        @pl.when(wid < rem)
        def _():
            chunk(wid + 32 * nfull)

    return k(t_hm, t_q, t_pos, src2, dst2)


def _edge_body(ghs, gps, ghd, gqd, gpd, eattr, ecl, ete,
               Wce, Wde, Wae, eb, etW, etb,
               maskp, Wge, Wgi, Wgj, cg, cb,
               klg, klb, k2W, k2b,
               vlg, vlb, v2W, v2b,
               offs, Hm, HmT, Hm2,
               w1out, w2out):
    hi = ghs[...]
    hj = ghd[...]
    qd = gqd[...]
    psx = gps[...]
    pdx = gpd[...]

    dp = (psx - pdx) * maskp[...]
    d2 = jnp.sum(dp * dp, axis=-1, keepdims=True)
    dist = jnp.sqrt(d2 + 1e-12)
    df = jnp.exp(_COEFF * (dist - offs[...]) ** 2)

    ea = _mm(ecl[...], Wce[...]) + _mm(df, Wde[...]) + _mm(eattr[...], Wae[...]) + eb[...]
    et = _mm(_silu(ete[...]), etW[...]) + etb[...]
    esh = et[:, :EDGE_DIM]
    esc = et[:, EDGE_DIM:]
    eam = _lnorm(ea, 1e-6) * (1.0 + esc) + esh

    D = 2 * NODE_DIM + EDGE_DIM
    s = (jnp.sum(eam, axis=-1, keepdims=True)
         + psx[:, 3:4] + pdx[:, 3:4])
    ss = (jnp.sum(eam * eam, axis=-1, keepdims=True)
          + psx[:, 4:5] + pdx[:, 4:5])
    mu = s / D
    var = ss / D - mu * mu
    inv = lax.rsqrt(var + 1e-5)

    xmm = _mmb(eam, Wge[...]) + _mmb(hi, Wgi[...]) + _mmb(hj, Wgj[...])
    kv1 = xmm * inv - (mu * inv) * cg[...] + cb[...]
    k1 = kv1[:, :NODE_DIM]
    v1 = kv1[:, NODE_DIM:]

    kk = _silu(_lnorm(k1, 1e-5) * klg[...] + klb[...])
    kv = _mmb(kk, k2W[...]) + k2b[...]

    vv = _silu(_lnorm(v1, 1e-5) * vlg[...] + vlb[...])
    v = _mmb(vv, v2W[...]) + v2b[...]

    lg = _mm(qd * kv, Hm[...]) * 0.25
    ex = jnp.exp(lg)
    exb = _mm(ex, HmT[...])
    w1out[...] = exb * v
    w2out[...] = _mm(ex, Hm2[...])


def _edge(ghs, gps, ghd, gqd, gpd, edge_attr, clash_feat, ete, p):
    E = ghs.shape[0]
    B = 2000
    r1 = lambda a: a.reshape(1, -1)
    Wkv = jnp.concatenate([p['k1_W'], p['v1_W']], axis=1)
    kvb = jnp.concatenate([p['k1_b'], p['v1_b']]).reshape(1, -1)
    lng = p['lin_norm_g']
    lnb = p['lin_norm_b']
    Wg = (Wkv * lng[:, None]).astype(jnp.bfloat16)
    cg = r1(lng @ Wkv)
    cb = r1(lnb @ Wkv) + kvb
    maskp = np.zeros((1, 16), np.float32); maskp[0, :3] = 1.0
    args = (ghs, gps, ghd, gqd, gpd, edge_attr, clash_feat, ete,
            p['edge_emb_W'][:16], p['edge_emb_W'][16:80], p['edge_emb_W'][80:96],
            r1(p['edge_emb_b']), p['edge_time_W'], r1(p['edge_time_b']),
            maskp, Wg[:16], Wg[16:144], Wg[144:272], cg, cb,
            r1(p['k_ln_g']), r1(p['k_ln_b']),
            p['k2_W'].astype(jnp.bfloat16), r1(p['k2_b']),
            r1(p['v_ln_g']), r1(p['v_ln_b']),
            p['v2_W'].astype(jnp.bfloat16), r1(p['v2_b']),
            _OFFS.reshape(1, -1), _HM, _HMT, _HM2)
    row = lambda c: pl.BlockSpec((B, c), lambda i: (i, 0))
    full = lambda a: pl.BlockSpec(jnp.shape(a), lambda i: (0,) * jnp.ndim(a))
    in_specs = [row(128), row(16), row(128), row(128), row(16),
                row(16), row(16), row(128)] + [full(a) for a in args[8:]]
    return pl.pallas_call(
        _edge_body,
        grid=(E // B,),
        in_specs=in_specs,
        out_specs=[row(128), row(16)],
        out_shape=[jax.ShapeDtypeStruct((E, 128), F32),
                   jax.ShapeDtypeStruct((E, 16), F32)],
    )(*args)


def _sc_scatter(w1, w2, dst2, z1, z2, N):
    E = w1.shape[0]
    nchunks = E // 128
    nfull = nchunks // 32
    rem = nchunks - nfull * 32
    rpt = N // 16
    mesh = plsc.VectorSubcoreMesh(core_axis_name="c", subcore_axis_name="s")

    @functools.partial(
        pl.kernel,
        out_type=[jax.ShapeDtypeStruct((2 * N, 128), F32),
                  jax.ShapeDtypeStruct((2 * N, 16), F32)],
        mesh=mesh,
        scratch_types=[pltpu.VMEM((128,), jnp.int32),
                       pltpu.VMEM((128, 128), F32),
                       pltpu.VMEM((128, 16), F32),
                       pltpu.VMEM_SHARED((N, 128), F32),
                       pltpu.VMEM_SHARED((N, 16), F32)],
        compiler_params=pltpu.CompilerParams(use_tc_tiling_on_sc=False),
    )
    def k(wv1, wv2, d2, zz1, zz2, outA, outB, idx_v, w1v, w2v, tabA, tabB):
        c = lax.axis_index("c")
        s = lax.axis_index("s")
        wid = s * 2 + c
        r0 = s * rpt
        pltpu.sync_copy(zz1, tabA.at[pl.ds(r0, rpt)])
        pltpu.sync_copy(zz2, tabB.at[pl.ds(r0, rpt)])
        plsc.subcore_barrier()

        def chunk(ci):
            pltpu.sync_copy(d2.at[ci], idx_v)
            pltpu.sync_copy(wv1.at[pl.ds(ci * 128, 128)], w1v)
            pltpu.sync_copy(wv2.at[pl.ds(ci * 128, 128)], w2v)
            pltpu.sync_copy(w1v, tabA.at[idx_v], add=True)
            pltpu.sync_copy(w2v, tabB.at[idx_v], add=True)

        def body(j, carry):
            chunk(wid + 32 * j)
            return carry

        lax.fori_loop(0, nfull, body, 0)

        @pl.when(wid < rem)
        def _():
            chunk(wid + 32 * nfull)

        plsc.subcore_barrier()
        pltpu.sync_copy(tabA.at[pl.ds(r0, rpt)], outA.at[pl.ds(c * N + r0, rpt)])
        pltpu.sync_copy(tabB.at[pl.ds(r0, rpt)], outB.at[pl.ds(c * N + r0, rpt)])

    return k(w1, w2, dst2, z1, z2)


def _node2_body(pA0, pA1, pB0, pB1, h, o1W, o1b, o2W, o2b,
                f1W, f1b, f2W, f2b, HmT, hout):
    num = pA0[...] + pA1[...]
    den = pB0[:, :HEADS] + pB1[:, :HEADS]
    deb = _mm(den, HmT[...])
    out = num / (deb + 1e-16)
    o = _mm(_silu(_mm(out, o1W[...]) + o1b[...]), o2W[...]) + o2b[...]
    f = _mm(_silu(_mm(o, f1W[...]) + f1b[...]), f2W[...]) + f2b[...]
    hout[...] = h[...] + f


def _node2(pA0, pA1, pB0, pB1, h, p):
    N = h.shape[0]
    B = 1000
    r1 = lambda a: a.reshape(1, -1)
    args = (pA0, pA1, pB0, pB1, h,
            p['out1_W'], r1(p['out1_b']), p['out2_W'], r1(p['out2_b']),
            p['ff1_W'], r1(p['ff1_b']), p['ff2_W'], r1(p['ff2_b']), _HMT)
    row = lambda c: pl.BlockSpec((B, c), lambda i: (i, 0))
    full = lambda a: pl.BlockSpec(jnp.shape(a), lambda i: (0,) * jnp.ndim(a))
    in_specs = [row(128), row(128), row(16), row(16), row(128)] + \
               [full(a) for a in args[5:]]
    return pl.pallas_call(
        _node2_body,
        grid=(N // B,),
        in_specs=in_specs,
        out_specs=row(128),
        out_shape=jax.ShapeDtypeStruct((N, 128), F32),
    )(*args)


def kernel(pos, h, edge_attr, clash_feat, edge_index, node_time_emb,
           edge_time_emb, params):
    N = h.shape[0]
    E = edge_attr.shape[0]
    src2 = edge_index[0].reshape(E // 128, 128)
    dst2 = edge_index[1].reshape(E // 128, 128)
    pos16 = jnp.pad(pos, ((0, 0), (0, 13)))

    t_hm, t_q, t_px = _node1(h, node_time_emb, pos16, params)
    ghs, gps, ghd, gqd, gpd = _sc_gather(t_hm, t_q, t_px, src2, dst2)
    w1, w2 = _edge(ghs, gps, ghd, gqd, gpd, edge_attr, clash_feat,
                   edge_time_emb, params)
    z1 = jnp.zeros((N // 16, 128), F32)
    z2 = jnp.zeros((N // 16, 16), F32)
    partsA, partsB = _sc_scatter(w1, w2, dst2, z1, z2, N)
    h_node = _node2(partsA[:N], partsA[N:], partsB[:N], partsB[N:], h, params)
    return (h_node, pos)


# speedup vs baseline: 21.4378x; 1.1301x over previous
import functools

import jax
import jax.numpy as jnp
import numpy as np
from jax import lax
from jax.experimental import pallas as pl
from jax.experimental.pallas import tpu as pltpu
from jax.experimental.pallas import tpu_sc as plsc

F32 = jnp.float32

NODE_DIM = 128
EDGE_DIM = 16
DIST_DIM = 64
HEADS = 8
HEAD_C = 16

_OFFS = np.linspace(0.0, 15.0, DIST_DIM).astype(np.float32)
_COEFF = float(-0.5 / (_OFFS[1] - _OFFS[0]) ** 2)
_HM = np.kron(np.eye(HEADS, dtype=np.float32), np.ones((HEAD_C, 1), np.float32))
_HMT = _HM.T.copy()
_HM2 = np.zeros((HEADS, 128), np.float32)
_HM2[:, :8] = np.eye(HEADS); _HM2[:, 8:16] = np.eye(HEADS)


def _silu(x):
    return x * (1.0 / (1.0 + jnp.exp(-x)))


def _lnorm(x, eps):
    mu = jnp.mean(x, axis=-1, keepdims=True)
    xc = x - mu
    var = jnp.mean(xc * xc, axis=-1, keepdims=True)
    return xc * lax.rsqrt(var + eps)


def _mm(a, b):
    return jnp.dot(a, b, preferred_element_type=F32)


def _mmb(a, b):
    return jnp.dot(a.astype(jnp.bfloat16), b, preferred_element_type=F32)


def _node1_body(h, nte, pos16, u3, u4, ntW, ntb, q1W, q1b, qlg, qlb, q2W, q2b,
                ohm, oq, opx):
    nt = _mm(_silu(nte[...]), ntW[...]) + ntb[...]
    nsh = nt[:, :NODE_DIM]
    nsc = nt[:, NODE_DIM:]
    hm = _lnorm(h[...], 1e-6) * (1.0 + nsc) + nsh
    qh = _silu(_lnorm(_mm(hm, q1W[...]) + q1b[...], 1e-5) * qlg[...] + qlb[...])
    oq[...] = _mm(qh, q2W[...]) + q2b[...]
    ohm[...] = hm
    s1 = jnp.sum(hm, axis=-1, keepdims=True)
    s2 = jnp.sum(hm * hm, axis=-1, keepdims=True)
    opx[...] = pos16[...] + s1 * u3[...] + s2 * u4[...]


def _node1(h, nte, pos16, p):
    N = h.shape[0]
    B = 1000
    u3 = np.zeros((1, 16), np.float32); u3[0, 3] = 1.0
    u4 = np.zeros((1, 16), np.float32); u4[0, 4] = 1.0
    full = lambda a: pl.BlockSpec(jnp.shape(a), lambda i: (0,) * jnp.ndim(a))
    row = lambda c: pl.BlockSpec((B, c), lambda i: (i, 0))
    args = (h, nte, pos16, u3, u4,
            p['node_time_W'], p['node_time_b'].reshape(1, -1),
            p['q1_W'], p['q1_b'].reshape(1, -1),
            p['q_ln_g'].reshape(1, -1), p['q_ln_b'].reshape(1, -1),
            p['q2_W'], p['q2_b'].reshape(1, -1))
    in_specs = [row(128), row(128), row(16)] + [full(a) for a in args[3:]]
    return pl.pallas_call(
        _node1_body,
        grid=(N // B,),
        in_specs=in_specs,
        out_specs=[row(128), row(128), row(16)],
        out_shape=[jax.ShapeDtypeStruct((N, 128), F32),
                   jax.ShapeDtypeStruct((N, 128), F32),
                   jax.ShapeDtypeStruct((N, 16), F32)],
    )(*args)


def _sc_gather(t_hm, t_q, t_pos, src2, dst2):
    E = src2.shape[0] * 128
    nchunks = E // 128
    nfull = nchunks // 32
    rem = nchunks - nfull * 32
    mesh = plsc.VectorSubcoreMesh(core_axis_name="c", subcore_axis_name="s")

    @functools.partial(
        pl.kernel,
        out_type=[jax.ShapeDtypeStruct((E, 128), F32),
                  jax.ShapeDtypeStruct((E, 128), F32),
                  jax.ShapeDtypeStruct((E, 128), F32),
                  jax.ShapeDtypeStruct((E, 128), F32),
                  jax.ShapeDtypeStruct((E, 128), F32)],
        mesh=mesh,
        scratch_types=[pltpu.VMEM((128,), jnp.int32),
                       pltpu.VMEM((128,), jnp.int32),
                       pltpu.VMEM((128, 128), F32),
                       pltpu.VMEM((128, 16), F32),
                       pltpu.VMEM((128, 128), F32),
                       pltpu.VMEM((128, 128), F32),
                       pltpu.VMEM((128, 16), F32),
                       pltpu.SemaphoreType.DMA,
                       pltpu.SemaphoreType.DMA,
                       pltpu.SemaphoreType.DMA,
                       pltpu.SemaphoreType.DMA,
                       pltpu.SemaphoreType.DMA],
        compiler_params=pltpu.CompilerParams(use_tc_tiling_on_sc=False),
    )
    def k(thm, tq, tpos, s2, d2, ghs, gps, ghd, gqd, gpd,
          idx_s, idx_d, r_hs, r_ps, r_hd, r_qd, r_pd,
          m1, m2, m3, m4, m5):
        wid = lax.axis_index("s") * 2 + lax.axis_index("c")

        def chunk(c):
            base = c * 128
            pltpu.sync_copy(s2.at[c], idx_s)
            pltpu.sync_copy(d2.at[c], idx_d)
            c1 = pltpu.async_copy(thm.at[idx_s], r_hs, m1)
            c2 = pltpu.async_copy(tpos.at[idx_s], r_ps, m2)
            c3 = pltpu.async_copy(thm.at[idx_d], r_hd, m3)
            c4 = pltpu.async_copy(tq.at[idx_d], r_qd, m4)
            c5 = pltpu.async_copy(tpos.at[idx_d], r_pd, m5)
            c1.wait(); c2.wait(); c3.wait(); c4.wait(); c5.wait()
            pltpu.sync_copy(r_hs, ghs.at[pl.ds(base, 128)])
            pltpu.sync_copy(r_ps, gps.at[pl.ds(base, 128), pl.ds(0, 16)])
            pltpu.sync_copy(r_hd, ghd.at[pl.ds(base, 128)])
            pltpu.sync_copy(r_qd, gqd.at[pl.ds(base, 128)])
            pltpu.sync_copy(r_pd, gpd.at[pl.ds(base, 128), pl.ds(0, 16)])

        def body(j, carry):
            chunk(wid + 32 * j)
            return carry

        lax.fori_loop(0, nfull, body, 0)

        @pl.when(wid < rem)
        def _():
            chunk(wid + 32 * nfull)

    return k(t_hm, t_q, t_pos, src2, dst2)


def _edge_body(ghs, gps, ghd, gqd, gpd, eattr, ecl, ete,
               Wce, Wde, Wae, eb, etW, etb,
               maskp, Wge, Wgi, Wgj, cg, cb,
               klg, klb, k2W, k2b,
               vlg, vlb, v2W, v2b,
               offs, Hm, HmT, Hm2,
               w1out, w2out):
    hi = ghs[...]
    hj = ghd[...]
    qd = gqd[...]
    psx = gps[...]
    pdx = gpd[...]

    dp = (psx - pdx) * maskp[...]
    d2 = jnp.sum(dp * dp, axis=-1, keepdims=True)
    dist = jnp.sqrt(d2 + 1e-12)
    df = jnp.exp(_COEFF * (dist - offs[...]) ** 2)

    ea = _mm(ecl[...], Wce[...]) + _mm(df, Wde[...]) + _mm(eattr[...], Wae[...]) + eb[...]
    et = _mm(_silu(ete[...]), etW[...]) + etb[...]
    esh = et[:, :EDGE_DIM]
    esc = et[:, EDGE_DIM:]
    eam = _lnorm(ea, 1e-6) * (1.0 + esc) + esh

    D = 2 * NODE_DIM + EDGE_DIM
    s = (jnp.sum(eam, axis=-1, keepdims=True)
         + psx[:, 3:4] + pdx[:, 3:4])
    ss = (jnp.sum(eam * eam, axis=-1, keepdims=True)
          + psx[:, 4:5] + pdx[:, 4:5])
    mu = s / D
    var = ss / D - mu * mu
    inv = lax.rsqrt(var + 1e-5)

    xmm = _mmb(eam, Wge[...]) + _mmb(hi, Wgi[...]) + _mmb(hj, Wgj[...])
    kv1 = xmm * inv - (mu * inv) * cg[...] + cb[...]
    k1 = kv1[:, :NODE_DIM]
    v1 = kv1[:, NODE_DIM:]

    kk = _silu(_lnorm(k1, 1e-5) * klg[...] + klb[...])
    kv = _mmb(kk, k2W[...]) + k2b[...]

    vv = _silu(_lnorm(v1, 1e-5) * vlg[...] + vlb[...])
    v = _mmb(vv, v2W[...]) + v2b[...]

    lg = _mm(qd * kv, Hm[...]) * 0.25
    ex = jnp.exp(lg)
    exb = _mm(ex, HmT[...])
    w1out[...] = exb * v
    w2out[...] = _mm(ex, Hm2[...])


def _edge(ghs, gps, ghd, gqd, gpd, edge_attr, clash_feat, ete, p):
    E = ghs.shape[0]
    B = 2000
    r1 = lambda a: a.reshape(1, -1)
    Wkv = jnp.concatenate([p['k1_W'], p['v1_W']], axis=1)
    kvb = jnp.concatenate([p['k1_b'], p['v1_b']]).reshape(1, -1)
    lng = p['lin_norm_g']
    lnb = p['lin_norm_b']
    Wg = (Wkv * lng[:, None]).astype(jnp.bfloat16)
    cg = r1(lng @ Wkv)
    cb = r1(lnb @ Wkv) + kvb
    maskp = np.zeros((1, 128), np.float32); maskp[0, :3] = 1.0
    args = (ghs, gps, ghd, gqd, gpd, edge_attr, clash_feat, ete,
            p['edge_emb_W'][:16], p['edge_emb_W'][16:80], p['edge_emb_W'][80:96],
            r1(p['edge_emb_b']), p['edge_time_W'], r1(p['edge_time_b']),
            maskp, Wg[:16], Wg[16:144], Wg[144:272], cg, cb,
            r1(p['k_ln_g']), r1(p['k_ln_b']),
            p['k2_W'].astype(jnp.bfloat16), r1(p['k2_b']),
            r1(p['v_ln_g']), r1(p['v_ln_b']),
            p['v2_W'].astype(jnp.bfloat16), r1(p['v2_b']),
            _OFFS.reshape(1, -1), _HM, _HMT, _HM2)
    row = lambda c: pl.BlockSpec((B, c), lambda i: (i, 0))
    full = lambda a: pl.BlockSpec(jnp.shape(a), lambda i: (0,) * jnp.ndim(a))
    in_specs = [row(128), row(128), row(128), row(128), row(128),
                row(16), row(16), row(128)] + [full(a) for a in args[8:]]
    return pl.pallas_call(
        _edge_body,
        grid=(E // B,),
        in_specs=in_specs,
        out_specs=[row(128), row(128)],
        out_shape=[jax.ShapeDtypeStruct((E, 128), F32),
                   jax.ShapeDtypeStruct((E, 128), F32)],
    )(*args)


def _sc_scatter(w1, w2, dst2, z1, z2, N):
    E = w1.shape[0]
    nchunks = E // 128
    nfull = nchunks // 32
    rem = nchunks - nfull * 32
    rpt = N // 16
    mesh = plsc.VectorSubcoreMesh(core_axis_name="c", subcore_axis_name="s")

    @functools.partial(
        pl.kernel,
        out_type=[jax.ShapeDtypeStruct((2 * N, 128), F32),
                  jax.ShapeDtypeStruct((2 * N, 16), F32)],
        mesh=mesh,
        scratch_types=[pltpu.VMEM((128,), jnp.int32),
                       pltpu.VMEM((128, 128), F32),
                       pltpu.VMEM((128, 16), F32),
                       pltpu.VMEM_SHARED((N, 128), F32),
                       pltpu.VMEM_SHARED((N, 16), F32)],
        compiler_params=pltpu.CompilerParams(use_tc_tiling_on_sc=False),
    )
    def k(wv1, wv2, d2, zz1, zz2, outA, outB, idx_v, w1v, w2v, tabA, tabB):
        c = lax.axis_index("c")
        s = lax.axis_index("s")
        wid = s * 2 + c
        r0 = s * rpt
        pltpu.sync_copy(zz1, tabA.at[pl.ds(r0, rpt)])
        pltpu.sync_copy(zz2, tabB.at[pl.ds(r0, rpt)])
        plsc.subcore_barrier()

        def chunk(ci):
            pltpu.sync_copy(d2.at[ci], idx_v)
            pltpu.sync_copy(wv1.at[pl.ds(ci * 128, 128)], w1v)
            pltpu.sync_copy(wv2.at[pl.ds(ci * 128, 128), pl.ds(0, 16)], w2v)
            pltpu.sync_copy(w1v, tabA.at[idx_v], add=True)
            pltpu.sync_copy(w2v, tabB.at[idx_v], add=True)

        def body(j, carry):
            chunk(wid + 32 * j)
            return carry

        lax.fori_loop(0, nfull, body, 0)

        @pl.when(wid < rem)
        def _():
            chunk(wid + 32 * nfull)

        plsc.subcore_barrier()
        pltpu.sync_copy(tabA.at[pl.ds(r0, rpt)], outA.at[pl.ds(c * N + r0, rpt)])
        pltpu.sync_copy(tabB.at[pl.ds(r0, rpt)], outB.at[pl.ds(c * N + r0, rpt)])

    return k(w1, w2, dst2, z1, z2)


def _node2_body(pA0, pA1, pB0, pB1, h, o1W, o1b, o2W, o2b,
                f1W, f1b, f2W, f2b, HmT, hout):
    num = pA0[...] + pA1[...]
    den = pB0[:, :HEADS] + pB1[:, :HEADS]
    deb = _mm(den, HmT[...])
    out = num / (deb + 1e-16)
    o = _mm(_silu(_mm(out, o1W[...]) + o1b[...]), o2W[...]) + o2b[...]
    f = _mm(_silu(_mm(o, f1W[...]) + f1b[...]), f2W[...]) + f2b[...]
    hout[...] = h[...] + f


def _node2(pA0, pA1, pB0, pB1, h, p):
    N = h.shape[0]
    B = 1000
    r1 = lambda a: a.reshape(1, -1)
    args = (pA0, pA1, pB0, pB1, h,
            p['out1_W'], r1(p['out1_b']), p['out2_W'], r1(p['out2_b']),
            p['ff1_W'], r1(p['ff1_b']), p['ff2_W'], r1(p['ff2_b']), _HMT)
    row = lambda c: pl.BlockSpec((B, c), lambda i: (i, 0))
    full = lambda a: pl.BlockSpec(jnp.shape(a), lambda i: (0,) * jnp.ndim(a))
    in_specs = [row(128), row(128), row(16), row(16), row(128)] + \
               [full(a) for a in args[5:]]
    return pl.pallas_call(
        _node2_body,
        grid=(N // B,),
        in_specs=in_specs,
        out_specs=row(128),
        out_shape=jax.ShapeDtypeStruct((N, 128), F32),
    )(*args)


def kernel(pos, h, edge_attr, clash_feat, edge_index, node_time_emb,
           edge_time_emb, params):
    N = h.shape[0]
    E = edge_attr.shape[0]
    src2 = edge_index[0].reshape(E // 128, 128)
    dst2 = edge_index[1].reshape(E // 128, 128)
    pos16 = jnp.pad(pos, ((0, 0), (0, 13)))

    t_hm, t_q, t_px = _node1(h, node_time_emb, pos16, params)
    ghs, gps, ghd, gqd, gpd = _sc_gather(t_hm, t_q, t_px, src2, dst2)
    w1, w2 = _edge(ghs, gps, ghd, gqd, gpd, edge_attr, clash_feat,
                   edge_time_emb, params)
    z1 = jnp.zeros((N // 16, 128), F32)
    z2 = jnp.zeros((N // 16, 16), F32)
    partsA, partsB = _sc_scatter(w1, w2, dst2, z1, z2, N)
    h_node = _node2(partsA[:N], partsA[N:], partsB[:N], partsB[N:], h, params)
    return (h_node, pos)


# speedup vs baseline: 21.7590x; 1.0150x over previous
import functools

import jax
import jax.numpy as jnp
import numpy as np
from jax import lax
from jax.experimental import pallas as pl
from jax.experimental.pallas import tpu as pltpu
from jax.experimental.pallas import tpu_sc as plsc

F32 = jnp.float32

NODE_DIM = 128
EDGE_DIM = 16
DIST_DIM = 64
HEADS = 8
HEAD_C = 16

_OFFS = np.linspace(0.0, 15.0, DIST_DIM).astype(np.float32)
_COEFF = float(-0.5 / (_OFFS[1] - _OFFS[0]) ** 2)
_HM = np.kron(np.eye(HEADS, dtype=np.float32), np.ones((HEAD_C, 1), np.float32))
_HMT = _HM.T.copy()
_HM2 = np.zeros((HEADS, 128), np.float32)
_HM2[:, :8] = np.eye(HEADS); _HM2[:, 8:16] = np.eye(HEADS)


def _silu(x):
    return x * (1.0 / (1.0 + jnp.exp(-x)))


def _lnorm(x, eps):
    mu = jnp.mean(x, axis=-1, keepdims=True)
    xc = x - mu
    var = jnp.mean(xc * xc, axis=-1, keepdims=True)
    return xc * lax.rsqrt(var + eps)


def _mm(a, b):
    return jnp.dot(a, b, preferred_element_type=F32)


def _mmb(a, b):
    return jnp.dot(a.astype(jnp.bfloat16), b, preferred_element_type=F32)


def _node1_body(h, nte, pos16, u3, u4, ntW, ntb, q1W, q1b, qlg, qlb, q2W, q2b,
                ohm, oq, opx):
    nt = _mm(_silu(nte[...]), ntW[...]) + ntb[...]
    nsh = nt[:, :NODE_DIM]
    nsc = nt[:, NODE_DIM:]
    hm = _lnorm(h[...], 1e-6) * (1.0 + nsc) + nsh
    qh = _silu(_lnorm(_mm(hm, q1W[...]) + q1b[...], 1e-5) * qlg[...] + qlb[...])
    oq[...] = _mm(qh, q2W[...]) + q2b[...]
    ohm[...] = hm
    s1 = jnp.sum(hm, axis=-1, keepdims=True)
    s2 = jnp.sum(hm * hm, axis=-1, keepdims=True)
    opx[...] = pos16[...] + s1 * u3[...] + s2 * u4[...]


def _node1(h, nte, pos16, p):
    N = h.shape[0]
    B = 1000
    u3 = np.zeros((1, 16), np.float32); u3[0, 3] = 1.0
    u4 = np.zeros((1, 16), np.float32); u4[0, 4] = 1.0
    full = lambda a: pl.BlockSpec(jnp.shape(a), lambda i: (0,) * jnp.ndim(a))
    row = lambda c: pl.BlockSpec((B, c), lambda i: (i, 0))
    args = (h, nte, pos16, u3, u4,
            p['node_time_W'], p['node_time_b'].reshape(1, -1),
            p['q1_W'], p['q1_b'].reshape(1, -1),
            p['q_ln_g'].reshape(1, -1), p['q_ln_b'].reshape(1, -1),
            p['q2_W'], p['q2_b'].reshape(1, -1))
    in_specs = [row(128), row(128), row(16)] + [full(a) for a in args[3:]]
    return pl.pallas_call(
        _node1_body,
        grid=(N // B,),
        in_specs=in_specs,
        out_specs=[row(128), row(128), row(16)],
        out_shape=[jax.ShapeDtypeStruct((N, 128), F32),
                   jax.ShapeDtypeStruct((N, 128), F32),
                   jax.ShapeDtypeStruct((N, 16), F32)],
    )(*args)


def _sc_gather(t_hm, t_q, t_pos, src2, dst2):
    E = src2.shape[0] * 128
    nchunks = E // 128
    nfull = nchunks // 32
    rem = nchunks - nfull * 32
    mesh = plsc.VectorSubcoreMesh(core_axis_name="c", subcore_axis_name="s")

    @functools.partial(
        pl.kernel,
        out_type=[jax.ShapeDtypeStruct((E, 128), F32),
                  jax.ShapeDtypeStruct((E, 128), F32),
                  jax.ShapeDtypeStruct((E, 128), F32),
                  jax.ShapeDtypeStruct((E, 128), F32),
                  jax.ShapeDtypeStruct((E, 128), F32)],
        mesh=mesh,
        scratch_types=[pltpu.VMEM((128,), jnp.int32),
                       pltpu.VMEM((128,), jnp.int32),
                       pltpu.VMEM((128, 128), F32),
                       pltpu.VMEM((128, 16), F32),
                       pltpu.VMEM((128, 128), F32),
                       pltpu.VMEM((128, 128), F32),
                       pltpu.VMEM((128, 16), F32),
                       pltpu.SemaphoreType.DMA,
                       pltpu.SemaphoreType.DMA,
                       pltpu.SemaphoreType.DMA,
                       pltpu.SemaphoreType.DMA,
                       pltpu.SemaphoreType.DMA],
        compiler_params=pltpu.CompilerParams(use_tc_tiling_on_sc=False),
    )
    def k(thm, tq, tpos, s2, d2, ghs, gps, ghd, gqd, gpd,
          idx_s, idx_d, r_hs, r_ps, r_hd, r_qd, r_pd,
          m1, m2, m3, m4, m5):
        wid = lax.axis_index("s") * 2 + lax.axis_index("c")

        def chunk(c):
            base = c * 128
            pltpu.sync_copy(s2.at[c], idx_s)
            pltpu.sync_copy(d2.at[c], idx_d)
            c1 = pltpu.async_copy(thm.at[idx_s], r_hs, m1)
            c2 = pltpu.async_copy(tpos.at[idx_s], r_ps, m2)
            c3 = pltpu.async_copy(thm.at[idx_d], r_hd, m3)
            c4 = pltpu.async_copy(tq.at[idx_d], r_qd, m4)
            c5 = pltpu.async_copy(tpos.at[idx_d], r_pd, m5)
            c1.wait(); c2.wait(); c3.wait(); c4.wait(); c5.wait()
            pltpu.sync_copy(r_hs, ghs.at[pl.ds(base, 128)])
            pltpu.sync_copy(r_ps, gps.at[pl.ds(base, 128), pl.ds(0, 16)])
            pltpu.sync_copy(r_hd, ghd.at[pl.ds(base, 128)])
            pltpu.sync_copy(r_qd, gqd.at[pl.ds(base, 128)])
            pltpu.sync_copy(r_pd, gpd.at[pl.ds(base, 128), pl.ds(0, 16)])

        def body(j, carry):
            chunk(wid + 32 * j)
            return carry

        lax.fori_loop(0, nfull, body, 0)

        @pl.when(wid < rem)
        def _():
            chunk(wid + 32 * nfull)

    return k(t_hm, t_q, t_pos, src2, dst2)


def _edge_body(ghs, gps, ghd, gqd, gpd, eattr, ecl, ete,
               Wce, Wde, Wae, eb, etW, etb,
               maskp, Wge, Wgi, Wgj, cg, cb,
               klg, klb, k2W, k2b,
               vlg, vlb, v2W, v2b,
               offs, Hm, HmT, Hm2,
               w1out, w2out):
    hi = ghs[...]
    hj = ghd[...]
    qd = gqd[...]
    psx = gps[...]
    pdx = gpd[...]

    dp = (psx - pdx) * maskp[...]
    d2 = jnp.sum(dp * dp, axis=-1, keepdims=True)
    dist = jnp.sqrt(d2 + 1e-12)
    df = jnp.exp(_COEFF * (dist - offs[...]) ** 2)

    ea = (_mm(ecl[...].T, Wce[...]) + _mm(df, Wde[...])
          + _mm(eattr[...].T, Wae[...]) + eb[...])
    et = _mm(_silu(ete[...]), etW[...]) + etb[...]
    esh = et[:, :EDGE_DIM]
    esc = et[:, EDGE_DIM:]
    eam = _lnorm(ea, 1e-6) * (1.0 + esc) + esh

    D = 2 * NODE_DIM + EDGE_DIM
    s = (jnp.sum(eam, axis=-1, keepdims=True)
         + psx[:, 3:4] + pdx[:, 3:4])
    ss = (jnp.sum(eam * eam, axis=-1, keepdims=True)
          + psx[:, 4:5] + pdx[:, 4:5])
    mu = s / D
    var = ss / D - mu * mu
    inv = lax.rsqrt(var + 1e-5)

    xmm = _mmb(eam, Wge[...]) + _mmb(hi, Wgi[...]) + _mmb(hj, Wgj[...])
    kv1 = xmm * inv - (mu * inv) * cg[...] + cb[...]
    k1 = kv1[:, :NODE_DIM]
    v1 = kv1[:, NODE_DIM:]

    kk = _silu(_lnorm(k1, 1e-5) * klg[...] + klb[...])
    kv = _mmb(kk, k2W[...]) + k2b[...]

    vv = _silu(_lnorm(v1, 1e-5) * vlg[...] + vlb[...])
    v = _mmb(vv, v2W[...]) + v2b[...]

    lg = _mm(qd * kv, Hm[...]) * 0.25
    ex = jnp.exp(lg)
    exb = _mm(ex, HmT[...])
    w1out[...] = exb * v
    w2out[...] = _mm(ex, Hm2[...])


def _edge(ghs, gps, ghd, gqd, gpd, edge_attr, clash_feat, ete, p):
    E = ghs.shape[0]
    B = 3200
    r1 = lambda a: a.reshape(1, -1)
    Wkv = jnp.concatenate([p['k1_W'], p['v1_W']], axis=1)
    kvb = jnp.concatenate([p['k1_b'], p['v1_b']]).reshape(1, -1)
    lng = p['lin_norm_g']
    lnb = p['lin_norm_b']
    Wg = (Wkv * lng[:, None]).astype(jnp.bfloat16)
    cg = r1(lng @ Wkv)
    cb = r1(lnb @ Wkv) + kvb
    maskp = np.zeros((1, 128), np.float32); maskp[0, :3] = 1.0
    args = (ghs, gps, ghd, gqd, gpd, edge_attr.T, clash_feat.T, ete,
            p['edge_emb_W'][:16], p['edge_emb_W'][16:80], p['edge_emb_W'][80:96],
            r1(p['edge_emb_b']), p['edge_time_W'], r1(p['edge_time_b']),
            maskp, Wg[:16], Wg[16:144], Wg[144:272], cg, cb,
            r1(p['k_ln_g']), r1(p['k_ln_b']),
            p['k2_W'].astype(jnp.bfloat16), r1(p['k2_b']),
            r1(p['v_ln_g']), r1(p['v_ln_b']),
            p['v2_W'].astype(jnp.bfloat16), r1(p['v2_b']),
            _OFFS.reshape(1, -1), _HM, _HMT, _HM2)
    row = lambda c: pl.BlockSpec((B, c), lambda i: (i, 0))
    colT = pl.BlockSpec((16, B), lambda i: (0, i))
    full = lambda a: pl.BlockSpec(jnp.shape(a), lambda i: (0,) * jnp.ndim(a))
    in_specs = [row(128), row(128), row(128), row(128), row(128),
                colT, colT, row(128)] + [full(a) for a in args[8:]]
    return pl.pallas_call(
        _edge_body,
        grid=(E // B,),
        in_specs=in_specs,
        out_specs=[row(128), row(128)],
        out_shape=[jax.ShapeDtypeStruct((E, 128), F32),
                   jax.ShapeDtypeStruct((E, 128), F32)],
    )(*args)


def _sc_scatter(w1, w2, dst2, z1, z2, N):
    E = w1.shape[0]
    nchunks = E // 128
    nfull = nchunks // 32
    rem = nchunks - nfull * 32
    rpt = N // 16
    mesh = plsc.VectorSubcoreMesh(core_axis_name="c", subcore_axis_name="s")

    @functools.partial(
        pl.kernel,
        out_type=[jax.ShapeDtypeStruct((2 * N, 128), F32),
                  jax.ShapeDtypeStruct((2 * N, 16), F32)],
        mesh=mesh,
        scratch_types=[pltpu.VMEM((128,), jnp.int32),
                       pltpu.VMEM((128, 128), F32),
                       pltpu.VMEM((128, 16), F32),
                       pltpu.VMEM_SHARED((N, 128), F32),
                       pltpu.VMEM_SHARED((N, 16), F32)],
        compiler_params=pltpu.CompilerParams(use_tc_tiling_on_sc=False),
    )
    def k(wv1, wv2, d2, zz1, zz2, outA, outB, idx_v, w1v, w2v, tabA, tabB):
        c = lax.axis_index("c")
        s = lax.axis_index("s")
        wid = s * 2 + c
        r0 = s * rpt
        pltpu.sync_copy(zz1, tabA.at[pl.ds(r0, rpt)])
        pltpu.sync_copy(zz2, tabB.at[pl.ds(r0, rpt)])
        plsc.subcore_barrier()

        def chunk(ci):
            pltpu.sync_copy(d2.at[ci], idx_v)
            pltpu.sync_copy(wv1.at[pl.ds(ci * 128, 128)], w1v)
            pltpu.sync_copy(wv2.at[pl.ds(ci * 128, 128), pl.ds(0, 16)], w2v)
            pltpu.sync_copy(w1v, tabA.at[idx_v], add=True)
            pltpu.sync_copy(w2v, tabB.at[idx_v], add=True)

        def body(j, carry):
            chunk(wid + 32 * j)
            return carry

        lax.fori_loop(0, nfull, body, 0)

        @pl.when(wid < rem)
        def _():
            chunk(wid + 32 * nfull)

        plsc.subcore_barrier()
        pltpu.sync_copy(tabA.at[pl.ds(r0, rpt)], outA.at[pl.ds(c * N + r0, rpt)])
        pltpu.sync_copy(tabB.at[pl.ds(r0, rpt)], outB.at[pl.ds(c * N + r0, rpt)])

    return k(w1, w2, dst2, z1, z2)


def _node2_body(pA0, pA1, pB0, pB1, h, o1W, o1b, o2W, o2b,
                f1W, f1b, f2W, f2b, HmT, hout):
    num = pA0[...] + pA1[...]
    den = pB0[:, :HEADS] + pB1[:, :HEADS]
    deb = _mm(den, HmT[...])
    out = num / (deb + 1e-16)
    o = _mm(_silu(_mm(out, o1W[...]) + o1b[...]), o2W[...]) + o2b[...]
    f = _mm(_silu(_mm(o, f1W[...]) + f1b[...]), f2W[...]) + f2b[...]
    hout[...] = h[...] + f


def _node2(pA0, pA1, pB0, pB1, h, p):
    N = h.shape[0]
    B = 1000
    r1 = lambda a: a.reshape(1, -1)
    args = (pA0, pA1, pB0, pB1, h,
            p['out1_W'], r1(p['out1_b']), p['out2_W'], r1(p['out2_b']),
            p['ff1_W'], r1(p['ff1_b']), p['ff2_W'], r1(p['ff2_b']), _HMT)
    row = lambda c: pl.BlockSpec((B, c), lambda i: (i, 0))
    full = lambda a: pl.BlockSpec(jnp.shape(a), lambda i: (0,) * jnp.ndim(a))
    in_specs = [row(128), row(128), row(16), row(16), row(128)] + \
               [full(a) for a in args[5:]]
    return pl.pallas_call(
        _node2_body,
        grid=(N // B,),
        in_specs=in_specs,
        out_specs=row(128),
        out_shape=jax.ShapeDtypeStruct((N, 128), F32),
    )(*args)


def kernel(pos, h, edge_attr, clash_feat, edge_index, node_time_emb,
           edge_time_emb, params):
    N = h.shape[0]
    E = edge_attr.shape[0]
    src2 = edge_index[0].reshape(E // 128, 128)
    dst2 = edge_index[1].reshape(E // 128, 128)
    pos16 = jnp.pad(pos, ((0, 0), (0, 13)))

    t_hm, t_q, t_px = _node1(h, node_time_emb, pos16, params)
    ghs, gps, ghd, gqd, gpd = _sc_gather(t_hm, t_q, t_px, src2, dst2)
    w1, w2 = _edge(ghs, gps, ghd, gqd, gpd, edge_attr, clash_feat,
                   edge_time_emb, params)
    z1 = jnp.zeros((N // 16, 128), F32)
    z2 = jnp.zeros((N // 16, 16), F32)
    partsA, partsB = _sc_scatter(w1, w2, dst2, z1, z2, N)
    h_node = _node2(partsA[:N], partsA[N:], partsB[:N], partsB[N:], h, params)
    return (h_node, pos)


# speedup vs baseline: 24.8142x; 1.1404x over previous
import functools

import jax
import jax.numpy as jnp
import numpy as np
from jax import lax
from jax.experimental import pallas as pl
from jax.experimental.pallas import tpu as pltpu
from jax.experimental.pallas import tpu_sc as plsc

F32 = jnp.float32

NODE_DIM = 128
EDGE_DIM = 16
DIST_DIM = 64
HEADS = 8
HEAD_C = 16

_OFFS = np.linspace(0.0, 15.0, DIST_DIM).astype(np.float32)
_COEFF = float(-0.5 / (_OFFS[1] - _OFFS[0]) ** 2)
_HM = np.kron(np.eye(HEADS, dtype=np.float32), np.ones((HEAD_C, 1), np.float32))
_HMT = _HM.T.copy()
_HM2 = np.zeros((HEADS, 128), np.float32)
_HM2[:, :8] = np.eye(HEADS); _HM2[:, 8:16] = np.eye(HEADS)


def _silu(x):
    return x * (1.0 / (1.0 + jnp.exp(-x)))


def _lnorm(x, eps):
    mu = jnp.mean(x, axis=-1, keepdims=True)
    xc = x - mu
    var = jnp.mean(xc * xc, axis=-1, keepdims=True)
    return xc * lax.rsqrt(var + eps)


def _mm(a, b):
    return jnp.dot(a, b, preferred_element_type=F32)


def _mmb(a, b):
    return jnp.dot(a.astype(jnp.bfloat16), b, preferred_element_type=F32)


def _node1_body(h, nte, pos16, u3, u4, ntW, ntb, q1W, q1b, qlg, qlb, q2W, q2b,
                ohm, oq, opx):
    nt = _mm(_silu(nte[...]), ntW[...]) + ntb[...]
    nsh = nt[:, :NODE_DIM]
    nsc = nt[:, NODE_DIM:]
    hm = _lnorm(h[...], 1e-6) * (1.0 + nsc) + nsh
    qh = _silu(_lnorm(_mm(hm, q1W[...]) + q1b[...], 1e-5) * qlg[...] + qlb[...])
    oq[...] = _mm(qh, q2W[...]) + q2b[...]
    ohm[...] = hm
    s1 = jnp.sum(hm, axis=-1, keepdims=True)
    s2 = jnp.sum(hm * hm, axis=-1, keepdims=True)
    opx[...] = pos16[...] + s1 * u3[...] + s2 * u4[...]


def _node1(h, nte, pos16, p):
    N = h.shape[0]
    B = 1000
    u3 = np.zeros((1, 16), np.float32); u3[0, 3] = 1.0
    u4 = np.zeros((1, 16), np.float32); u4[0, 4] = 1.0
    full = lambda a: pl.BlockSpec(jnp.shape(a), lambda i: (0,) * jnp.ndim(a))
    row = lambda c: pl.BlockSpec((B, c), lambda i: (i, 0))
    args = (h, nte, pos16, u3, u4,
            p['node_time_W'], p['node_time_b'].reshape(1, -1),
            p['q1_W'], p['q1_b'].reshape(1, -1),
            p['q_ln_g'].reshape(1, -1), p['q_ln_b'].reshape(1, -1),
            p['q2_W'], p['q2_b'].reshape(1, -1))
    in_specs = [row(128), row(128), row(16)] + [full(a) for a in args[3:]]
    return pl.pallas_call(
        _node1_body,
        grid=(N // B,),
        in_specs=in_specs,
        out_specs=[row(128), row(128), row(16)],
        out_shape=[jax.ShapeDtypeStruct((N, 128), F32),
                   jax.ShapeDtypeStruct((N, 128), F32),
                   jax.ShapeDtypeStruct((N, 16), F32)],
    )(*args)


def _sc_gather(t_hm, t_q, t_pos, src2, dst2):
    E = src2.shape[0] * 128
    nchunks = E // 128
    nfull = nchunks // 32
    rem = nchunks - nfull * 32
    mesh = plsc.VectorSubcoreMesh(core_axis_name="c", subcore_axis_name="s")

    @functools.partial(
        pl.kernel,
        out_type=[jax.ShapeDtypeStruct((E, 128), F32),
                  jax.ShapeDtypeStruct((E, 128), F32),
                  jax.ShapeDtypeStruct((E, 128), F32),
                  jax.ShapeDtypeStruct((E, 128), F32),
                  jax.ShapeDtypeStruct((E, 128), F32)],
        mesh=mesh,
        scratch_types=[pltpu.VMEM((128,), jnp.int32),
                       pltpu.VMEM((128,), jnp.int32),
                       pltpu.VMEM((128, 128), F32),
                       pltpu.VMEM((128, 16), F32),
                       pltpu.VMEM((128, 128), F32),
                       pltpu.VMEM((128, 128), F32),
                       pltpu.VMEM((128, 16), F32),
                       pltpu.SemaphoreType.DMA,
                       pltpu.SemaphoreType.DMA,
                       pltpu.SemaphoreType.DMA,
                       pltpu.SemaphoreType.DMA,
                       pltpu.SemaphoreType.DMA],
        compiler_params=pltpu.CompilerParams(use_tc_tiling_on_sc=False),
    )
    def k(thm, tq, tpos, s2, d2, ghs, gps, ghd, gqd, gpd,
          idx_s, idx_d, r_hs, r_ps, r_hd, r_qd, r_pd,
          m1, m2, m3, m4, m5):
        wid = lax.axis_index("s") * 2 + lax.axis_index("c")

        def chunk(c):
            base = c * 128
            pltpu.sync_copy(s2.at[c], idx_s)
            pltpu.sync_copy(d2.at[c], idx_d)
            c1 = pltpu.async_copy(thm.at[idx_s], r_hs, m1)
            c2 = pltpu.async_copy(tpos.at[idx_s], r_ps, m2)
            c3 = pltpu.async_copy(thm.at[idx_d], r_hd, m3)
            c4 = pltpu.async_copy(tq.at[idx_d], r_qd, m4)
            c5 = pltpu.async_copy(tpos.at[idx_d], r_pd, m5)
            c1.wait(); c2.wait(); c3.wait(); c4.wait(); c5.wait()
            pltpu.sync_copy(r_hs, ghs.at[pl.ds(base, 128)])
            pltpu.sync_copy(r_ps, gps.at[pl.ds(base, 128), pl.ds(0, 16)])
            pltpu.sync_copy(r_hd, ghd.at[pl.ds(base, 128)])
            pltpu.sync_copy(r_qd, gqd.at[pl.ds(base, 128)])
            pltpu.sync_copy(r_pd, gpd.at[pl.ds(base, 128), pl.ds(0, 16)])

        def body(j, carry):
            chunk(wid + 32 * j)
            return carry

        lax.fori_loop(0, nfull, body, 0)

        @pl.when(wid < rem)
        def _():
            chunk(wid + 32 * nfull)

    return k(t_hm, t_q, t_pos, src2, dst2)


def _edge_body(ghs, gps, ghd, gqd, gpd, eattr, ecl, ete,
               Wce, Wde, Wae, eb, etW, etb,
               maskp, Wge, Wgi, Wgj, cg, cb,
               klg, klb, k2W, k2b,
               vlg, vlb, v2W, v2b,
               offs, Hm, HmT, Hm2,
               w1out, w2out):
    hi = ghs[...]
    hj = ghd[...]
    qd = gqd[...]
    psx = gps[...]
    pdx = gpd[...]

    dp = (psx - pdx) * maskp[...]
    d2 = jnp.sum(dp * dp, axis=-1, keepdims=True)
    dist = jnp.sqrt(d2 + 1e-12)
    df = jnp.exp(_COEFF * (dist - offs[...]) ** 2)

    ea = (_mm(ecl[...].T, Wce[...]) + _mm(df, Wde[...])
          + _mm(eattr[...].T, Wae[...]) + eb[...])
    et = _mm(_silu(ete[...]), etW[...]) + etb[...]
    esh = et[:, :EDGE_DIM]
    esc = et[:, EDGE_DIM:]
    eam = _lnorm(ea, 1e-6) * (1.0 + esc) + esh

    D = 2 * NODE_DIM + EDGE_DIM
    s = (jnp.sum(eam, axis=-1, keepdims=True)
         + psx[:, 3:4] + pdx[:, 3:4])
    ss = (jnp.sum(eam * eam, axis=-1, keepdims=True)
          + psx[:, 4:5] + pdx[:, 4:5])
    mu = s / D
    var = ss / D - mu * mu
    inv = lax.rsqrt(var + 1e-5)

    xmm = _mmb(eam, Wge[...]) + _mmb(hi, Wgi[...]) + _mmb(hj, Wgj[...])
    kv1 = xmm * inv - (mu * inv) * cg[...] + cb[...]
    k1 = kv1[:, :NODE_DIM]
    v1 = kv1[:, NODE_DIM:]

    kk = _silu(_lnorm(k1, 1e-5) * klg[...] + klb[...])
    kv = _mmb(kk, k2W[...]) + k2b[...]

    vv = _silu(_lnorm(v1, 1e-5) * vlg[...] + vlb[...])
    v = _mmb(vv, v2W[...]) + v2b[...]

    lg = _mm(qd * kv, Hm[...]) * 0.25
    ex = jnp.exp(lg)
    exb = _mm(ex, HmT[...])
    w1out[...] = exb * v
    w2out[...] = _mm(ex, Hm2[...])


def _edge(ghs, gps, ghd, gqd, gpd, edge_attr_T, clash_feat_T, ete, p):
    E = ghs.shape[0]
    B = 3200
    r1 = lambda a: a.reshape(1, -1)
    Wkv = jnp.concatenate([p['k1_W'], p['v1_W']], axis=1)
    kvb = jnp.concatenate([p['k1_b'], p['v1_b']]).reshape(1, -1)
    lng = p['lin_norm_g']
    lnb = p['lin_norm_b']
    Wg = (Wkv * lng[:, None]).astype(jnp.bfloat16)
    cg = r1(lng @ Wkv)
    cb = r1(lnb @ Wkv) + kvb
    maskp = np.zeros((1, 128), np.float32); maskp[0, :3] = 1.0
    args = (ghs, gps, ghd, gqd, gpd, edge_attr_T, clash_feat_T, ete,
            p['edge_emb_W'][:16], p['edge_emb_W'][16:80], p['edge_emb_W'][80:96],
            r1(p['edge_emb_b']), p['edge_time_W'], r1(p['edge_time_b']),
            maskp, Wg[:16], Wg[16:144], Wg[144:272], cg, cb,
            r1(p['k_ln_g']), r1(p['k_ln_b']),
            p['k2_W'].astype(jnp.bfloat16), r1(p['k2_b']),
            r1(p['v_ln_g']), r1(p['v_ln_b']),
            p['v2_W'].astype(jnp.bfloat16), r1(p['v2_b']),
            _OFFS.reshape(1, -1), _HM, _HMT, _HM2)
    row = lambda c: pl.BlockSpec((B, c), lambda i: (i, 0))
    colT = pl.BlockSpec((16, B), lambda i: (0, i))
    full = lambda a: pl.BlockSpec(jnp.shape(a), lambda i: (0,) * jnp.ndim(a))
    in_specs = [row(128), row(128), row(128), row(128), row(128),
                colT, colT, row(128)] + [full(a) for a in args[8:]]
    return pl.pallas_call(
        _edge_body,
        grid=(E // B,),
        in_specs=in_specs,
        out_specs=[row(128), row(128)],
        out_shape=[jax.ShapeDtypeStruct((E, 128), F32),
                   jax.ShapeDtypeStruct((E, 128), F32)],
    )(*args)


def _sc_scatter(w1, w2, dst2, z1, z2, N):
    E = w1.shape[0]
    nchunks = E // 128
    nfull = nchunks // 32
    rem = nchunks - nfull * 32
    rpt = N // 16
    mesh = plsc.VectorSubcoreMesh(core_axis_name="c", subcore_axis_name="s")

    @functools.partial(
        pl.kernel,
        out_type=[jax.ShapeDtypeStruct((2 * N, 128), F32),
                  jax.ShapeDtypeStruct((2 * N, 16), F32)],
        mesh=mesh,
        scratch_types=[pltpu.VMEM((128,), jnp.int32),
                       pltpu.VMEM((128, 128), F32),
                       pltpu.VMEM((128, 16), F32),
                       pltpu.VMEM_SHARED((N, 128), F32),
                       pltpu.VMEM_SHARED((N, 16), F32)],
        compiler_params=pltpu.CompilerParams(use_tc_tiling_on_sc=False),
    )
    def k(wv1, wv2, d2, zz1, zz2, outA, outB, idx_v, w1v, w2v, tabA, tabB):
        c = lax.axis_index("c")
        s = lax.axis_index("s")
        wid = s * 2 + c
        r0 = s * rpt
        pltpu.sync_copy(zz1, tabA.at[pl.ds(r0, rpt)])
        pltpu.sync_copy(zz2, tabB.at[pl.ds(r0, rpt)])
        plsc.subcore_barrier()

        def chunk(ci):
            pltpu.sync_copy(d2.at[ci], idx_v)
            pltpu.sync_copy(wv1.at[pl.ds(ci * 128, 128)], w1v)
            pltpu.sync_copy(wv2.at[pl.ds(ci * 128, 128), pl.ds(0, 16)], w2v)
            pltpu.sync_copy(w1v, tabA.at[idx_v], add=True)
            pltpu.sync_copy(w2v, tabB.at[idx_v], add=True)

        def body(j, carry):
            chunk(wid + 32 * j)
            return carry

        lax.fori_loop(0, nfull, body, 0)

        @pl.when(wid < rem)
        def _():
            chunk(wid + 32 * nfull)

        plsc.subcore_barrier()
        pltpu.sync_copy(tabA.at[pl.ds(r0, rpt)], outA.at[pl.ds(c * N + r0, rpt)])
        pltpu.sync_copy(tabB.at[pl.ds(r0, rpt)], outB.at[pl.ds(c * N + r0, rpt)])

    return k(w1, w2, dst2, z1, z2)


def _node2_body(pA0, pA1, pA2, pA3, pB0, pB1, pB2, pB3, h,
                o1W, o1b, o2W, o2b, f1W, f1b, f2W, f2b, HmT, hout):
    num = (pA0[...] + pA1[...]) + (pA2[...] + pA3[...])
    den = ((pB0[:, :HEADS] + pB1[:, :HEADS])
           + (pB2[:, :HEADS] + pB3[:, :HEADS]))
    deb = _mm(den, HmT[...])
    out = num / (deb + 1e-16)
    o = _mm(_silu(_mm(out, o1W[...]) + o1b[...]), o2W[...]) + o2b[...]
    f = _mm(_silu(_mm(o, f1W[...]) + f1b[...]), f2W[...]) + f2b[...]
    hout[...] = h[...] + f


def _node2(pAs, pBs, h, p):
    N = h.shape[0]
    B = 1000
    r1 = lambda a: a.reshape(1, -1)
    args = tuple(pAs) + tuple(pBs) + (h,
            p['out1_W'], r1(p['out1_b']), p['out2_W'], r1(p['out2_b']),
            p['ff1_W'], r1(p['ff1_b']), p['ff2_W'], r1(p['ff2_b']), _HMT)
    row = lambda c: pl.BlockSpec((B, c), lambda i: (i, 0))
    full = lambda a: pl.BlockSpec(jnp.shape(a), lambda i: (0,) * jnp.ndim(a))
    in_specs = [row(128)] * 4 + [row(16)] * 4 + [row(128)] + \
               [full(a) for a in args[9:]]
    return pl.pallas_call(
        _node2_body,
        grid=(N // B,),
        in_specs=in_specs,
        out_specs=row(128),
        out_shape=jax.ShapeDtypeStruct((N, 128), F32),
    )(*args)


def kernel(pos, h, edge_attr, clash_feat, edge_index, node_time_emb,
           edge_time_emb, params):
    N = h.shape[0]
    E = edge_attr.shape[0]
    src2 = edge_index[0].reshape(E // 128, 128)
    dst2 = edge_index[1].reshape(E // 128, 128)
    pos16 = jnp.pad(pos, ((0, 0), (0, 13)))

    t_hm, t_q, t_px = _node1(h, node_time_emb, pos16, params)
    eaT = edge_attr.T
    clT = clash_feat.T
    z1 = jnp.zeros((N // 16, 128), F32)
    z2 = jnp.zeros((N // 16, 16), F32)

    Eh = E // 2
    nc2 = E // 256
    pAs, pBs = [], []
    for sl in range(2):
        lo = sl * Eh
        ga = _sc_gather(t_hm, t_q, t_px,
                        src2[sl * nc2:(sl + 1) * nc2],
                        dst2[sl * nc2:(sl + 1) * nc2])
        w1, w2 = _edge(ga[0], ga[1], ga[2], ga[3], ga[4],
                       eaT[:, lo:lo + Eh], clT[:, lo:lo + Eh],
                       edge_time_emb[lo:lo + Eh], params)
        pA, pB = _sc_scatter(w1, w2, dst2[sl * nc2:(sl + 1) * nc2],
                             z1, z2, N)
        pAs += [pA[:N], pA[N:]]
        pBs += [pB[:N], pB[N:]]

    h_node = _node2(pAs, pBs, h, params)
    return (h_node, pos)


# speedup vs baseline: 26.7717x; 1.0789x over previous
import functools

import jax
import jax.numpy as jnp
import numpy as np
from jax import lax
from jax.experimental import pallas as pl
from jax.experimental.pallas import tpu as pltpu
from jax.experimental.pallas import tpu_sc as plsc

F32 = jnp.float32

NODE_DIM = 128
EDGE_DIM = 16
DIST_DIM = 64
HEADS = 8
HEAD_C = 16

_OFFS = np.linspace(0.0, 15.0, DIST_DIM).astype(np.float32)
_COEFF = float(-0.5 / (_OFFS[1] - _OFFS[0]) ** 2)
_HM = np.kron(np.eye(HEADS, dtype=np.float32), np.ones((HEAD_C, 1), np.float32))
_HMT = _HM.T.copy()
_HM2 = np.zeros((HEADS, 128), np.float32)
_HM2[:, :8] = np.eye(HEADS); _HM2[:, 8:16] = np.eye(HEADS)


def _silu(x):
    return x * (1.0 / (1.0 + jnp.exp(-x)))


def _lnorm(x, eps):
    mu = jnp.mean(x, axis=-1, keepdims=True)
    xc = x - mu
    var = jnp.mean(xc * xc, axis=-1, keepdims=True)
    return xc * lax.rsqrt(var + eps)


def _mm(a, b):
    return jnp.dot(a, b, preferred_element_type=F32)


def _mmb(a, b):
    return jnp.dot(a.astype(jnp.bfloat16), b, preferred_element_type=F32)


def _node1_body(h, nte, pos16, u3, u4, ntW, ntb, q1W, q1b, qlg, qlb, q2W, q2b,
                ohm, oq, opx):
    nt = _mm(_silu(nte[...]), ntW[...]) + ntb[...]
    nsh = nt[:, :NODE_DIM]
    nsc = nt[:, NODE_DIM:]
    hm = _lnorm(h[...], 1e-6) * (1.0 + nsc) + nsh
    qh = _silu(_lnorm(_mm(hm, q1W[...]) + q1b[...], 1e-5) * qlg[...] + qlb[...])
    oq[...] = _mm(qh, q2W[...]) + q2b[...]
    ohm[...] = hm
    s1 = jnp.sum(hm, axis=-1, keepdims=True)
    s2 = jnp.sum(hm * hm, axis=-1, keepdims=True)
    opx[...] = pos16[...] + s1 * u3[...] + s2 * u4[...]


def _node1(h, nte, pos16, p):
    N = h.shape[0]
    B = 1000
    u3 = np.zeros((1, 16), np.float32); u3[0, 3] = 1.0
    u4 = np.zeros((1, 16), np.float32); u4[0, 4] = 1.0
    full = lambda a: pl.BlockSpec(jnp.shape(a), lambda i: (0,) * jnp.ndim(a))
    row = lambda c: pl.BlockSpec((B, c), lambda i: (i, 0))
    args = (h, nte, pos16, u3, u4,
            p['node_time_W'], p['node_time_b'].reshape(1, -1),
            p['q1_W'], p['q1_b'].reshape(1, -1),
            p['q_ln_g'].reshape(1, -1), p['q_ln_b'].reshape(1, -1),
            p['q2_W'], p['q2_b'].reshape(1, -1))
    in_specs = [row(128), row(128), row(16)] + [full(a) for a in args[3:]]
    return pl.pallas_call(
        _node1_body,
        grid=(N // B,),
        in_specs=in_specs,
        out_specs=[row(128), row(128), row(16)],
        out_shape=[jax.ShapeDtypeStruct((N, 128), F32),
                   jax.ShapeDtypeStruct((N, 128), F32),
                   jax.ShapeDtypeStruct((N, 16), F32)],
    )(*args)


def _sc_gather(t_hm, t_q, t_pos, src2, dst2):
    E = src2.shape[0] * 128
    nchunks = E // 128
    nfull = nchunks // 32
    rem = nchunks - nfull * 32
    mesh = plsc.VectorSubcoreMesh(core_axis_name="c", subcore_axis_name="s")

    @functools.partial(
        pl.kernel,
        out_type=[jax.ShapeDtypeStruct((E, 128), F32),
                  jax.ShapeDtypeStruct((E, 128), F32),
                  jax.ShapeDtypeStruct((E, 128), F32),
                  jax.ShapeDtypeStruct((E, 128), F32),
                  jax.ShapeDtypeStruct((E, 128), F32)],
        mesh=mesh,
        scratch_types=[pltpu.VMEM((128,), jnp.int32),
                       pltpu.VMEM((128,), jnp.int32),
                       pltpu.VMEM((128, 128), F32),
                       pltpu.VMEM((128, 16), F32),
                       pltpu.VMEM((128, 128), F32),
                       pltpu.VMEM((128, 128), F32),
                       pltpu.VMEM((128, 16), F32),
                       pltpu.SemaphoreType.DMA,
                       pltpu.SemaphoreType.DMA,
                       pltpu.SemaphoreType.DMA,
                       pltpu.SemaphoreType.DMA,
                       pltpu.SemaphoreType.DMA],
        compiler_params=pltpu.CompilerParams(use_tc_tiling_on_sc=False),
    )
    def k(thm, tq, tpos, s2, d2, ghs, gps, ghd, gqd, gpd,
          idx_s, idx_d, r_hs, r_ps, r_hd, r_qd, r_pd,
          m1, m2, m3, m4, m5):
        wid = lax.axis_index("s") * 2 + lax.axis_index("c")

        def chunk(c):
            base = c * 128
            pltpu.sync_copy(s2.at[c], idx_s)
            pltpu.sync_copy(d2.at[c], idx_d)
            c1 = pltpu.async_copy(thm.at[idx_s], r_hs, m1)
            c2 = pltpu.async_copy(tpos.at[idx_s], r_ps, m2)
            c3 = pltpu.async_copy(thm.at[idx_d], r_hd, m3)
            c4 = pltpu.async_copy(tq.at[idx_d], r_qd, m4)
            c5 = pltpu.async_copy(tpos.at[idx_d], r_pd, m5)
            c1.wait(); c2.wait(); c3.wait(); c4.wait(); c5.wait()
            pltpu.sync_copy(r_hs, ghs.at[pl.ds(base, 128)])
            pltpu.sync_copy(r_ps, gps.at[pl.ds(base, 128), pl.ds(0, 16)])
            pltpu.sync_copy(r_hd, ghd.at[pl.ds(base, 128)])
            pltpu.sync_copy(r_qd, gqd.at[pl.ds(base, 128)])
            pltpu.sync_copy(r_pd, gpd.at[pl.ds(base, 128), pl.ds(0, 16)])

        def body(j, carry):
            chunk(wid + 32 * j)
            return carry

        lax.fori_loop(0, nfull, body, 0)

        @pl.when(wid < rem)
        def _():
            chunk(wid + 32 * nfull)

    return k(t_hm, t_q, t_pos, src2, dst2)


def _edge_body(ghs, gps, ghd, gqd, gpd, eattr, ecl, ete,
               Wce, Wde, Wae, eb, etW, etb,
               maskp, Wge, Wgi, Wgj, cg, cb,
               klg, klb, k2W, k2b,
               vlg, vlb, v2W, v2b,
               offs, Hm, HmT, Hm2,
               w1out, w2out):
    hi = ghs[...]
    hj = ghd[...]
    qd = gqd[...]
    psx = gps[...]
    pdx = gpd[...]

    dp = (psx - pdx) * maskp[...]
    d2 = jnp.sum(dp * dp, axis=-1, keepdims=True)
    dist = jnp.sqrt(d2 + 1e-12)
    df = jnp.exp(_COEFF * (dist - offs[...]) ** 2)

    ea = (_mm(ecl[...].T, Wce[...]) + _mm(df, Wde[...])
          + _mm(eattr[...].T, Wae[...]) + eb[...])
    et = _mm(_silu(ete[...]), etW[...]) + etb[...]
    esh = et[:, :EDGE_DIM]
    esc = et[:, EDGE_DIM:]
    eam = _lnorm(ea, 1e-6) * (1.0 + esc) + esh

    D = 2 * NODE_DIM + EDGE_DIM
    s = (jnp.sum(eam, axis=-1, keepdims=True)
         + psx[:, 3:4] + pdx[:, 3:4])
    ss = (jnp.sum(eam * eam, axis=-1, keepdims=True)
          + psx[:, 4:5] + pdx[:, 4:5])
    mu = s / D
    var = ss / D - mu * mu
    inv = lax.rsqrt(var + 1e-5)

    xmm = _mmb(eam, Wge[...]) + _mmb(hi, Wgi[...]) + _mmb(hj, Wgj[...])
    kv1 = xmm * inv - (mu * inv) * cg[...] + cb[...]
    k1 = kv1[:, :NODE_DIM]
    v1 = kv1[:, NODE_DIM:]

    kk = _silu(_lnorm(k1, 1e-5) * klg[...] + klb[...])
    kv = _mmb(kk, k2W[...]) + k2b[...]

    vv = _silu(_lnorm(v1, 1e-5) * vlg[...] + vlb[...])
    v = _mmb(vv, v2W[...]) + v2b[...]

    lg = _mm(qd * kv, Hm[...]) * 0.25
    ex = jnp.exp(lg)
    exb = _mm(ex, HmT[...])
    w1out[...] = exb * v
    w2out[...] = _mm(ex, Hm2[...])


def _edge(ghs, gps, ghd, gqd, gpd, edge_attr_T, clash_feat_T, ete, p, toff):
    E = ghs.shape[0]
    B = 3200
    r1 = lambda a: a.reshape(1, -1)
    Wkv = jnp.concatenate([p['k1_W'], p['v1_W']], axis=1)
    kvb = jnp.concatenate([p['k1_b'], p['v1_b']]).reshape(1, -1)
    lng = p['lin_norm_g']
    lnb = p['lin_norm_b']
    Wg = (Wkv * lng[:, None]).astype(jnp.bfloat16)
    cg = r1(lng @ Wkv)
    cb = r1(lnb @ Wkv) + kvb
    maskp = np.zeros((1, 128), np.float32); maskp[0, :3] = 1.0
    args = (ghs, gps, ghd, gqd, gpd, edge_attr_T, clash_feat_T, ete,
            p['edge_emb_W'][:16], p['edge_emb_W'][16:80], p['edge_emb_W'][80:96],
            r1(p['edge_emb_b']), p['edge_time_W'], r1(p['edge_time_b']),
            maskp, Wg[:16], Wg[16:144], Wg[144:272], cg, cb,
            r1(p['k_ln_g']), r1(p['k_ln_b']),
            p['k2_W'].astype(jnp.bfloat16), r1(p['k2_b']),
            r1(p['v_ln_g']), r1(p['v_ln_b']),
            p['v2_W'].astype(jnp.bfloat16), r1(p['v2_b']),
            _OFFS.reshape(1, -1), _HM, _HMT, _HM2)
    row = lambda c: pl.BlockSpec((B, c), lambda i: (i, 0))
    rowo = pl.BlockSpec((B, 128), lambda i: (i + toff, 0))
    colT = pl.BlockSpec((16, B), lambda i: (0, i + toff))
    full = lambda a: pl.BlockSpec(jnp.shape(a), lambda i: (0,) * jnp.ndim(a))
    in_specs = [row(128), row(128), row(128), row(128), row(128),
                colT, colT, rowo] + [full(a) for a in args[8:]]
    return pl.pallas_call(
        _edge_body,
        grid=(E // B,),
        in_specs=in_specs,
        out_specs=[row(128), row(128)],
        out_shape=[jax.ShapeDtypeStruct((E, 128), F32),
                   jax.ShapeDtypeStruct((E, 128), F32)],
    )(*args)


def _sc_scatter(w1, w2, dst2, z1, z2, N):
    E = w1.shape[0]
    nchunks = E // 128
    nfull = nchunks // 32
    rem = nchunks - nfull * 32
    rpt = N // 16
    mesh = plsc.VectorSubcoreMesh(core_axis_name="c", subcore_axis_name="s")

    @functools.partial(
        pl.kernel,
        out_type=[jax.ShapeDtypeStruct((2 * N, 128), F32),
                  jax.ShapeDtypeStruct((2 * N, 16), F32)],
        mesh=mesh,
        scratch_types=[pltpu.VMEM((128,), jnp.int32),
                       pltpu.VMEM((128, 128), F32),
                       pltpu.VMEM((128, 16), F32),
                       pltpu.VMEM_SHARED((N, 128), F32),
                       pltpu.VMEM_SHARED((N, 16), F32)],
        compiler_params=pltpu.CompilerParams(use_tc_tiling_on_sc=False),
    )
    def k(wv1, wv2, d2, zz1, zz2, outA, outB, idx_v, w1v, w2v, tabA, tabB):
        c = lax.axis_index("c")
        s = lax.axis_index("s")
        wid = s * 2 + c
        r0 = s * rpt
        pltpu.sync_copy(zz1, tabA.at[pl.ds(r0, rpt)])
        pltpu.sync_copy(zz2, tabB.at[pl.ds(r0, rpt)])
        plsc.subcore_barrier()

        def chunk(ci):
            pltpu.sync_copy(d2.at[ci], idx_v)
            pltpu.sync_copy(wv1.at[pl.ds(ci * 128, 128)], w1v)
            pltpu.sync_copy(wv2.at[pl.ds(ci * 128, 128), pl.ds(0, 16)], w2v)
            pltpu.sync_copy(w1v, tabA.at[idx_v], add=True)
            pltpu.sync_copy(w2v, tabB.at[idx_v], add=True)

        def body(j, carry):
            chunk(wid + 32 * j)
            return carry

        lax.fori_loop(0, nfull, body, 0)

        @pl.when(wid < rem)
        def _():
            chunk(wid + 32 * nfull)

        plsc.subcore_barrier()
        pltpu.sync_copy(tabA.at[pl.ds(r0, rpt)], outA.at[pl.ds(c * N + r0, rpt)])
        pltpu.sync_copy(tabB.at[pl.ds(r0, rpt)], outB.at[pl.ds(c * N + r0, rpt)])

    return k(w1, w2, dst2, z1, z2)


def _node2_body(pA0, pA1, pA2, pA3, pB0, pB1, pB2, pB3, h,
                o1W, o1b, o2W, o2b, f1W, f1b, f2W, f2b, HmT, hout):
    num = (pA0[...] + pA1[...]) + (pA2[...] + pA3[...])
    den = ((pB0[:, :HEADS] + pB1[:, :HEADS])
           + (pB2[:, :HEADS] + pB3[:, :HEADS]))
    deb = _mm(den, HmT[...])
    out = num / (deb + 1e-16)
    o = _mm(_silu(_mm(out, o1W[...]) + o1b[...]), o2W[...]) + o2b[...]
    f = _mm(_silu(_mm(o, f1W[...]) + f1b[...]), f2W[...]) + f2b[...]
    hout[...] = h[...] + f


def _node2(pAs, pBs, h, p):
    N = h.shape[0]
    B = 1000
    nb = N // B
    r1 = lambda a: a.reshape(1, -1)
    args = (pAs[0], pAs[0], pAs[1], pAs[1],
            pBs[0], pBs[0], pBs[1], pBs[1], h,
            p['out1_W'], r1(p['out1_b']), p['out2_W'], r1(p['out2_b']),
            p['ff1_W'], r1(p['ff1_b']), p['ff2_W'], r1(p['ff2_b']), _HMT)
    row = lambda c: pl.BlockSpec((B, c), lambda i: (i, 0))
    row2 = lambda c: pl.BlockSpec((B, c), lambda i: (i + nb, 0))
    full = lambda a: pl.BlockSpec(jnp.shape(a), lambda i: (0,) * jnp.ndim(a))
    in_specs = [row(128), row2(128), row(128), row2(128),
                row(16), row2(16), row(16), row2(16), row(128)] + \
               [full(a) for a in args[9:]]
    return pl.pallas_call(
        _node2_body,
        grid=(N // B,),
        in_specs=in_specs,
        out_specs=row(128),
        out_shape=jax.ShapeDtypeStruct((N, 128), F32),
    )(*args)


def kernel(pos, h, edge_attr, clash_feat, edge_index, node_time_emb,
           edge_time_emb, params):
    N = h.shape[0]
    E = edge_attr.shape[0]
    src2 = edge_index[0].reshape(E // 128, 128)
    dst2 = edge_index[1].reshape(E // 128, 128)
    pos16 = jnp.pad(pos, ((0, 0), (0, 13)))

    t_hm, t_q, t_px = _node1(h, node_time_emb, pos16, params)
    eaT = edge_attr.T
    clT = clash_feat.T
    z1 = jnp.zeros((N // 16, 128), F32)
    z2 = jnp.zeros((N // 16, 16), F32)

    Eh = E // 2
    nc2 = E // 256
    pAs, pBs = [], []
    for sl in range(2):
        ga = _sc_gather(t_hm, t_q, t_px,
                        src2[sl * nc2:(sl + 1) * nc2],
                        dst2[sl * nc2:(sl + 1) * nc2])
        w1, w2 = _edge(ga[0], ga[1], ga[2], ga[3], ga[4],
                       eaT, clT, edge_time_emb, params,
                       sl * (Eh // 3200))
        pA, pB = _sc_scatter(w1, w2, dst2[sl * nc2:(sl + 1) * nc2],
                             z1, z2, N)
        pAs.append(pA)
        pBs.append(pB)

    h_node = _node2(pAs, pBs, h, params)
    return (h_node, pos)


# speedup vs baseline: 55.2656x; 2.0643x over previous
import functools

import jax
import jax.numpy as jnp
import numpy as np
from jax import lax
from jax.experimental import pallas as pl
from jax.experimental.pallas import tpu as pltpu
from jax.experimental.pallas import tpu_sc as plsc

F32 = jnp.float32

NODE_DIM = 128
EDGE_DIM = 16
DIST_DIM = 64
HEADS = 8
HEAD_C = 16

_OFFS = np.linspace(0.0, 15.0, DIST_DIM).astype(np.float32)
_COEFF = float(-0.5 / (_OFFS[1] - _OFFS[0]) ** 2)
_HM = np.kron(np.eye(HEADS, dtype=np.float32), np.ones((HEAD_C, 1), np.float32))
_HMT = _HM.T.copy()
_HM2 = np.zeros((HEADS, 128), np.float32)
_HM2[:, :8] = np.eye(HEADS); _HM2[:, 8:16] = np.eye(HEADS)


def _silu(x):
    return x * (1.0 / (1.0 + jnp.exp(-x)))


def _lnorm(x, eps):
    mu = jnp.mean(x, axis=-1, keepdims=True)
    xc = x - mu
    var = jnp.mean(xc * xc, axis=-1, keepdims=True)
    return xc * lax.rsqrt(var + eps)


def _mm(a, b):
    return jnp.dot(a, b, preferred_element_type=F32)


def _mmb(a, b):
    return jnp.dot(a.astype(jnp.bfloat16), b, preferred_element_type=F32)


def _node1_body(h, nte, pos16, u3, u4, ntW, ntb, q1W, q1b, qlg, qlb, q2W, q2b,
                ohm, oq, opx):
    nt = _mm(_silu(nte[...]), ntW[...]) + ntb[...]
    nsh = nt[:, :NODE_DIM]
    nsc = nt[:, NODE_DIM:]
    hm = _lnorm(h[...], 1e-6) * (1.0 + nsc) + nsh
    qh = _silu(_lnorm(_mm(hm, q1W[...]) + q1b[...], 1e-5) * qlg[...] + qlb[...])
    oq[...] = _mm(qh, q2W[...]) + q2b[...]
    ohm[...] = hm
    s1 = jnp.sum(hm, axis=-1, keepdims=True)
    s2 = jnp.sum(hm * hm, axis=-1, keepdims=True)
    opx[...] = pos16[...] + s1 * u3[...] + s2 * u4[...]


def _node1(h, nte, pos16, p):
    N = h.shape[0]
    B = 1000
    u3 = np.zeros((1, 16), np.float32); u3[0, 3] = 1.0
    u4 = np.zeros((1, 16), np.float32); u4[0, 4] = 1.0
    full = lambda a: pl.BlockSpec(jnp.shape(a), lambda i: (0,) * jnp.ndim(a))
    row = lambda c: pl.BlockSpec((B, c), lambda i: (i, 0))
    args = (h, nte, pos16, u3, u4,
            p['node_time_W'], p['node_time_b'].reshape(1, -1),
            p['q1_W'], p['q1_b'].reshape(1, -1),
            p['q_ln_g'].reshape(1, -1), p['q_ln_b'].reshape(1, -1),
            p['q2_W'], p['q2_b'].reshape(1, -1))
    in_specs = [row(128), row(128), row(16)] + [full(a) for a in args[3:]]
    return pl.pallas_call(
        _node1_body,
        grid=(N // B,),
        in_specs=in_specs,
        out_specs=[row(128), row(128), row(16)],
        out_shape=[jax.ShapeDtypeStruct((N, 128), F32),
                   jax.ShapeDtypeStruct((N, 128), F32),
                   jax.ShapeDtypeStruct((N, 16), F32)],
    )(*args)


def _sc_gather(t_hm, t_q, t_pos, src2, dst2):
    E = src2.shape[0] * 128
    nchunks = E // 128
    nfull = nchunks // 32
    rem = nchunks - nfull * 32
    mesh = plsc.VectorSubcoreMesh(core_axis_name="c", subcore_axis_name="s")

    @functools.partial(
        pl.kernel,
        out_type=[jax.ShapeDtypeStruct((E, 128), F32),
                  jax.ShapeDtypeStruct((E, 128), F32),
                  jax.ShapeDtypeStruct((E, 128), F32),
                  jax.ShapeDtypeStruct((E, 128), F32),
                  jax.ShapeDtypeStruct((E, 128), F32)],
        mesh=mesh,
        scratch_types=[pltpu.VMEM((128,), jnp.int32),
                       pltpu.VMEM((128,), jnp.int32),
                       pltpu.VMEM((128, 128), F32),
                       pltpu.VMEM((128, 16), F32),
                       pltpu.VMEM((128, 128), F32),
                       pltpu.VMEM((128, 128), F32),
                       pltpu.VMEM((128, 16), F32),
                       pltpu.SemaphoreType.DMA,
                       pltpu.SemaphoreType.DMA,
                       pltpu.SemaphoreType.DMA,
                       pltpu.SemaphoreType.DMA,
                       pltpu.SemaphoreType.DMA],
        compiler_params=pltpu.CompilerParams(use_tc_tiling_on_sc=False),
    )
    def k(thm, tq, tpos, s2, d2, ghs, gps, ghd, gqd, gpd,
          idx_s, idx_d, r_hs, r_ps, r_hd, r_qd, r_pd,
          m1, m2, m3, m4, m5):
        wid = lax.axis_index("s") * 2 + lax.axis_index("c")

        def chunk(c):
            base = c * 128
            pltpu.sync_copy(s2.at[c], idx_s)
            pltpu.sync_copy(d2.at[c], idx_d)
            c1 = pltpu.async_copy(thm.at[idx_s], r_hs, m1)
            c2 = pltpu.async_copy(tpos.at[idx_s], r_ps, m2)
            c3 = pltpu.async_copy(thm.at[idx_d], r_hd, m3)
            c4 = pltpu.async_copy(tq.at[idx_d], r_qd, m4)
            c5 = pltpu.async_copy(tpos.at[idx_d], r_pd, m5)
            c1.wait(); c2.wait(); c3.wait(); c4.wait(); c5.wait()
            pltpu.sync_copy(r_hs, ghs.at[pl.ds(base, 128)])
            pltpu.sync_copy(r_ps, gps.at[pl.ds(base, 128), pl.ds(0, 16)])
            pltpu.sync_copy(r_hd, ghd.at[pl.ds(base, 128)])
            pltpu.sync_copy(r_qd, gqd.at[pl.ds(base, 128)])
            pltpu.sync_copy(r_pd, gpd.at[pl.ds(base, 128), pl.ds(0, 16)])

        def body(j, carry):
            chunk(wid + 32 * j)
            return carry

        lax.fori_loop(0, nfull, body, 0)

        @pl.when(wid < rem)
        def _():
            chunk(wid + 32 * nfull)

    return k(t_hm, t_q, t_pos, src2, dst2)


def _edge_body(ghs, gps, ghd, gqd, gpd, eattr, ecl, ete,
               Wce, Wde, Wae, eb, etW, etb,
               maskp, Wge, Wgi, Wgj, cg, cb,
               klg, klb, k2W, k2b,
               vlg, vlb, v2W, v2b,
               offs, Hm, HmT, Hm2,
               w1out, w2out):
    hi = ghs[...]
    hj = ghd[...]
    qd = gqd[...]
    psx = gps[...]
    pdx = gpd[...]

    dp = (psx - pdx) * maskp[...]
    d2 = jnp.sum(dp * dp, axis=-1, keepdims=True)
    dist = jnp.sqrt(d2 + 1e-12)
    df = jnp.exp(_COEFF * (dist - offs[...]) ** 2)

    ea = (_mm(ecl[...].T, Wce[...]) + _mm(df, Wde[...])
          + _mm(eattr[...].T, Wae[...]) + eb[...])
    et = _mm(_silu(ete[...]), etW[...]) + etb[...]
    esh = et[:, :EDGE_DIM]
    esc = et[:, EDGE_DIM:]
    eam = _lnorm(ea, 1e-6) * (1.0 + esc) + esh

    D = 2 * NODE_DIM + EDGE_DIM
    s = (jnp.sum(eam, axis=-1, keepdims=True)
         + psx[:, 3:4] + pdx[:, 3:4])
    ss = (jnp.sum(eam * eam, axis=-1, keepdims=True)
          + psx[:, 4:5] + pdx[:, 4:5])
    mu = s / D
    var = ss / D - mu * mu
    inv = lax.rsqrt(var + 1e-5)

    xmm = _mmb(eam, Wge[...]) + _mmb(hi, Wgi[...]) + _mmb(hj, Wgj[...])
    kv1 = xmm * inv - (mu * inv) * cg[...] + cb[...]
    k1 = kv1[:, :NODE_DIM]
    v1 = kv1[:, NODE_DIM:]

    kk = _silu(_lnorm(k1, 1e-5) * klg[...] + klb[...])
    kv = _mmb(kk, k2W[...]) + k2b[...]

    vv = _silu(_lnorm(v1, 1e-5) * vlg[...] + vlb[...])
    v = _mmb(vv, v2W[...]) + v2b[...]

    lg = _mm(qd * kv, Hm[...]) * 0.25
    ex = jnp.exp(lg)
    exb = _mm(ex, HmT[...])
    w1out[...] = exb * v
    w2out[...] = _mm(ex, Hm2[...])


def _edge(ghs, gps, ghd, gqd, gpd, edge_attr_T, clash_feat_T, ete, p, toff):
    E = ghs.shape[0]
    B = 3200
    r1 = lambda a: a.reshape(1, -1)
    Wkv = jnp.concatenate([p['k1_W'], p['v1_W']], axis=1)
    kvb = jnp.concatenate([p['k1_b'], p['v1_b']]).reshape(1, -1)
    lng = p['lin_norm_g']
    lnb = p['lin_norm_b']
    Wg = (Wkv * lng[:, None]).astype(jnp.bfloat16)
    cg = r1(lng @ Wkv)
    cb = r1(lnb @ Wkv) + kvb
    maskp = np.zeros((1, 128), np.float32); maskp[0, :3] = 1.0
    args = (ghs, gps, ghd, gqd, gpd, edge_attr_T, clash_feat_T, ete,
            p['edge_emb_W'][:16], p['edge_emb_W'][16:80], p['edge_emb_W'][80:96],
            r1(p['edge_emb_b']), p['edge_time_W'], r1(p['edge_time_b']),
            maskp, Wg[:16], Wg[16:144], Wg[144:272], cg, cb,
            r1(p['k_ln_g']), r1(p['k_ln_b']),
            p['k2_W'].astype(jnp.bfloat16), r1(p['k2_b']),
            r1(p['v_ln_g']), r1(p['v_ln_b']),
            p['v2_W'].astype(jnp.bfloat16), r1(p['v2_b']),
            _OFFS.reshape(1, -1), _HM, _HMT, _HM2)
    row = lambda c: pl.BlockSpec((B, c), lambda i: (i, 0))
    rowo = pl.BlockSpec((B, 128), lambda i: (i + toff, 0))
    colT = pl.BlockSpec((16, B), lambda i: (0, i + toff))
    full = lambda a: pl.BlockSpec(jnp.shape(a), lambda i: (0,) * jnp.ndim(a))
    in_specs = [row(128), row(128), row(128), row(128), row(128),
                colT, colT, rowo] + [full(a) for a in args[8:]]
    return pl.pallas_call(
        _edge_body,
        grid=(E // B,),
        in_specs=in_specs,
        out_specs=[row(128), row(128)],
        out_shape=[jax.ShapeDtypeStruct((E, 128), F32),
                   jax.ShapeDtypeStruct((E, 128), F32)],
    )(*args)


def _sc_scatter(w1, w2, dst2, z1, z2, N):
    E = w1.shape[0]
    nchunks = E // 128
    nfull = nchunks // 32
    rem = nchunks - nfull * 32
    rpt = N // 16
    mesh = plsc.VectorSubcoreMesh(core_axis_name="c", subcore_axis_name="s")

    @functools.partial(
        pl.kernel,
        out_type=[jax.ShapeDtypeStruct((2 * N, 128), F32),
                  jax.ShapeDtypeStruct((2 * N, 16), F32)],
        mesh=mesh,
        scratch_types=[pltpu.VMEM((128,), jnp.int32),
                       pltpu.VMEM((128, 128), F32),
                       pltpu.VMEM((128, 16), F32),
                       pltpu.VMEM_SHARED((N, 128), F32),
                       pltpu.VMEM_SHARED((N, 16), F32)],
        compiler_params=pltpu.CompilerParams(use_tc_tiling_on_sc=False),
    )
    def k(wv1, wv2, d2, zz1, zz2, outA, outB, idx_v, w1v, w2v, tabA, tabB):
        c = lax.axis_index("c")
        s = lax.axis_index("s")
        wid = s * 2 + c
        r0 = s * rpt
        pltpu.sync_copy(zz1, tabA.at[pl.ds(r0, rpt)])
        pltpu.sync_copy(zz2, tabB.at[pl.ds(r0, rpt)])
        plsc.subcore_barrier()

        def chunk(ci):
            pltpu.sync_copy(d2.at[ci], idx_v)
            pltpu.sync_copy(wv1.at[pl.ds(ci * 128, 128)], w1v)
            pltpu.sync_copy(wv2.at[pl.ds(ci * 128, 128), pl.ds(0, 16)], w2v)
            pltpu.sync_copy(w1v, tabA.at[idx_v], add=True)
            pltpu.sync_copy(w2v, tabB.at[idx_v], add=True)

        def body(j, carry):
            chunk(wid + 32 * j)
            return carry

        lax.fori_loop(0, nfull, body, 0)

        @pl.when(wid < rem)
        def _():
            chunk(wid + 32 * nfull)

        plsc.subcore_barrier()
        pltpu.sync_copy(tabA.at[pl.ds(r0, rpt)], outA.at[pl.ds(c * N + r0, rpt)])
        pltpu.sync_copy(tabB.at[pl.ds(r0, rpt)], outB.at[pl.ds(c * N + r0, rpt)])

    return k(w1, w2, dst2, z1, z2)


def _node2_body(pA0, pA1, pA2, pA3, pB0, pB1, pB2, pB3, h,
                o1W, o1b, o2W, o2b, f1W, f1b, f2W, f2b, HmT, hout):
    num = (pA0[...] + pA1[...]) + (pA2[...] + pA3[...])
    den = ((pB0[:, :HEADS] + pB1[:, :HEADS])
           + (pB2[:, :HEADS] + pB3[:, :HEADS]))
    deb = _mm(den, HmT[...])
    out = num / (deb + 1e-16)
    o = _mm(_silu(_mm(out, o1W[...]) + o1b[...]), o2W[...]) + o2b[...]
    f = _mm(_silu(_mm(o, f1W[...]) + f1b[...]), f2W[...]) + f2b[...]
    hout[...] = h[...] + f


def _node2(pAs, pBs, h, p):
    N = h.shape[0]
    B = 1000
    nb = N // B
    r1 = lambda a: a.reshape(1, -1)
    args = (pAs[0], pAs[0], pAs[1], pAs[1],
            pBs[0], pBs[0], pBs[1], pBs[1], h,
            p['out1_W'], r1(p['out1_b']), p['out2_W'], r1(p['out2_b']),
            p['ff1_W'], r1(p['ff1_b']), p['ff2_W'], r1(p['ff2_b']), _HMT)
    row = lambda c: pl.BlockSpec((B, c), lambda i: (i, 0))
    row2 = lambda c: pl.BlockSpec((B, c), lambda i: (i + nb, 0))
    full = lambda a: pl.BlockSpec(jnp.shape(a), lambda i: (0,) * jnp.ndim(a))
    in_specs = [row(128), row2(128), row(128), row2(128),
                row(16), row2(16), row(16), row2(16), row(128)] + \
               [full(a) for a in args[9:]]
    return pl.pallas_call(
        _node2_body,
        grid=(N // B,),
        in_specs=in_specs,
        out_specs=row(128),
        out_shape=jax.ShapeDtypeStruct((N, 128), F32),
    )(*args)


def kernel(pos, h, edge_attr, clash_feat, edge_index, node_time_emb,
           edge_time_emb, params):
    N = h.shape[0]
    E = edge_attr.shape[0]
    src2 = edge_index[0].reshape(E // 128, 128)
    dst2 = edge_index[1].reshape(E // 128, 128)
    pos16 = jnp.pad(pos, ((0, 0), (0, 13)))

    t_hm, t_q, t_px = _node1(h, node_time_emb, pos16, params)
    eaT = edge_attr.T
    clT = clash_feat.T
    z1 = jnp.zeros((N // 16, 128), F32)
    z2 = jnp.zeros((N // 16, 16), F32)

    NSL = 5
    Eh = E // NSL
    nc2 = Eh // 128
    pAs, pBs = [], []
    for sl in range(NSL):
        ga = _sc_gather(t_hm, t_q, t_px,
                        src2[sl * nc2:(sl + 1) * nc2],
                        dst2[sl * nc2:(sl + 1) * nc2])
        w1, w2 = _edge(ga[0], ga[1], ga[2], ga[3], ga[4],
                       eaT, clT, edge_time_emb, params,
                       sl * (Eh // 3200))
        pA, pB = _sc_scatter(w1, w2, dst2[sl * nc2:(sl + 1) * nc2],
                             z1, z2, N)
        pAs.append(pA)
        pBs.append(pB)

    h_node = _node2(pAs, pBs, h, params)
    return (h_node, pos)
